# Initial kernel scaffold; baseline (speedup 1.0000x reference)
import jax
import jax.numpy as jnp
from jax.experimental import pallas as pl


def kernel(x, last_update, edge_index, t, msg, W_time, b_time, W_enc, b_enc, Wq, bq, Wk, bk, Wv, bv, We, be, W_asym, b_asym):
    raise NotImplementedError("write your pallas kernel here")


# speedup vs baseline: 4.0136x; 4.0136x over previous
import functools

import jax
import jax.numpy as jnp
from jax import lax
from jax.experimental import pallas as pl
from jax.experimental.pallas import tpu as pltpu
from jax.experimental.pallas import tpu_sc as plsc

N = 10000
E = 320000
MEM = 128
GAMMA = 0.1
EPSILON = 1.0
INV_SQRT_MEM = 1.0 / (128.0 ** 0.5)

K_D = 144
QE_D = 176
SM_D = 48
V_D = 64

NC = 2
NS = 16
NW = NC * NS
EPW = E // NW
C = 80
NCH = EPW // C
G = C // 16
NPAD = 10240
RPT = NPAD // NS
CPR = 128
NCP = RPT // CPR
EPT2 = E // NS
NR2 = EPT2 // EPW

BR = 200
GRID = N // BR

TWO_PI = 6.283185307179586
INV_2PI = 1.0 / TWO_PI
COS_COEF = (0.9999999999193508, -19.739208758208584, 64.93939011340913,
            -85.45668538180254, 60.24246470872289, -26.406761080377983,
            7.806608463960106, -1.4609479689305238)


def _cos_poly(z):
    u = z * INV_2PI
    n = u.astype(jnp.int32).astype(jnp.float32)
    f = u - n
    f = jnp.where(f > 0.5, f - 1.0, f)
    f = jnp.where(f < -0.5, f + 1.0, f)
    y = f * f
    acc = jnp.full_like(y, COS_COEF[7])
    for coef in COS_COEF[6::-1]:
        acc = acc * y + coef
    return acc


def _pre_body(x_ref, lu_ref, W_enc_ref, b_enc_ref, Wq_ref, bq_ref, Wk_ref,
              bk_ref, Wv_ref, bv_ref, We_ref, be_col_ref,
              k_ref, vlo_ref, vhi_ref, qe_ref, enc_ref):
    f32 = jnp.float32
    x = x_ref[...]
    dotT = lambda a, w: lax.dot_general(a, w, (((1,), (1,)), ((), ())),
                                        preferred_element_type=f32)
    enc = dotT(x, W_enc_ref[...]) + b_enc_ref[...]
    q = dotT(enc, Wq_ref[...]) + bq_ref[...]
    k = dotT(enc, Wk_ref[...]) + bk_ref[...]
    v = dotT(enc, Wv_ref[...]) + bv_ref[...]
    em = jnp.dot(q, We_ref[...], preferred_element_type=f32)
    qb = jnp.dot(q, be_col_ref[...], preferred_element_type=f32)
    pad = jnp.zeros((BR, 15), dtype=f32)
    k_ref[...] = jnp.concatenate([k, lu_ref[...], pad], axis=1)
    vlo_ref[...] = v[:, 0:V_D]
    vhi_ref[...] = v[:, V_D:MEM]
    qe_ref[...] = jnp.concatenate([q, em, qb, pad], axis=1)
    enc_ref[...] = enc


def _run_pre(x, lu2, W_enc, b_enc, Wq, bq, Wk, bk, Wv, bv, We, be_col):
    f32 = jnp.float32
    row = lambda d: pl.BlockSpec((BR, d), lambda i: (i, 0))
    full = lambda a, b: pl.BlockSpec((a, b), lambda i: (0, 0))
    return pl.pallas_call(
        _pre_body,
        grid=(GRID,),
        in_specs=[row(MEM), row(1), full(MEM, MEM), full(1, MEM),
                  full(MEM, MEM), full(1, MEM), full(MEM, MEM), full(1, MEM),
                  full(MEM, MEM), full(1, MEM), full(MEM, 32), full(MEM, 1)],
        out_specs=[row(K_D), row(V_D), row(V_D), row(QE_D), row(MEM)],
        out_shape=[jax.ShapeDtypeStruct((N, K_D), f32),
                   jax.ShapeDtypeStruct((N, V_D), f32),
                   jax.ShapeDtypeStruct((N, V_D), f32),
                   jax.ShapeDtypeStruct((N, QE_D), f32),
                   jax.ShapeDtypeStruct((N, MEM), f32)],
    )(x, lu2, W_enc, b_enc, Wq, bq, Wk, bk, Wv, bv, We, be_col)


def _alpha_body(k_hbm, qe_hbm, src_hbm, dst_hbm, t_hbm, msg_hbm, wt_hbm,
                bt_hbm, ex_hbm, out_hbm,
                srcv, dstv, kvv, qev, msgv, tv, exv, outv, wtv, btv, zb,
                accum, sem):
    c = lax.axis_index("c")
    s = lax.axis_index("s")
    wid = s * NC + c

    pltpu.sync_copy(wt_hbm, wtv)
    pltpu.sync_copy(bt_hbm, btv)
    wt = wtv[0, pl.ds(0, 16)]
    bt = btv[0, pl.ds(0, 16)]
    lane = lax.iota(jnp.int32, 16)
    unit = jnp.where(lane == 0, 1.0, 0.0).astype(jnp.float32)
    _dn = lax.GatherDimensionNumbers(offset_dims=(), collapsed_slice_dims=(0,),
                                     start_index_map=(0,))
    _perms = [(lane ^ m)[:, None] for m in (8, 4, 2, 1)]

    def _allsum(a):
        for p in _perms:
            a = a + lax.gather(a, p, _dn, slice_sizes=(1,),
                               mode=lax.GatherScatterMode.PROMISE_IN_BOUNDS)
        return a

    zeros16 = jnp.zeros((16,), jnp.float32)

    def zrow(j, carry):
        for kk in range(SM_D // 16):
            zb[j, pl.ds(kk * 16, 16)] = zeros16
        return carry

    lax.fori_loop(0, CPR, zrow, 0)
    for j in range(NCP):
        pltpu.sync_copy(zb, accum.at[pl.ds(s * RPT + j * CPR, CPR)])
    plsc.subcore_barrier()

    def chunk(i, carry):
        base = i * C
        pltpu.sync_copy(src_hbm.at[wid, pl.ds(base, C)], srcv)
        pltpu.sync_copy(dst_hbm.at[wid, pl.ds(base, C)], dstv)
        pltpu.sync_copy(t_hbm.at[wid, pl.ds(base, C)], tv.at[pl.ds(0, C)])
        pltpu.sync_copy(msg_hbm.at[pl.ds(wid * EPW + base, C)], msgv)
        pltpu.async_copy(k_hbm.at[srcv], kvv, sem).wait()
        pltpu.async_copy(qe_hbm.at[dstv], qev, sem).wait()

        def group(g, gcarry):
            exg = zeros16
            for j in range(16):
                e = g * 16 + j
                acc = qev[e, pl.ds(0, 16)] * kvv[e, pl.ds(0, 16)]
                for r in range(1, 8):
                    acc = acc + qev[e, pl.ds(16 * r, 16)] * kvv[e, pl.ds(16 * r, 16)]
                msg_v = msgv[e, pl.ds(0, 16)]
                lu = kvv[e, pl.ds(128, 16)][0]
                t_e = tv[pl.ds(e, 16)][0]
                rel = jnp.abs(lu - t_e)
                te = _cos_poly(rel * wt + bt)
                acc = acc + qev[e, pl.ds(128, 16)] * msg_v
                acc = acc + qev[e, pl.ds(144, 16)] * te
                acc = acc + qev[e, pl.ds(160, 16)]
                ex = jnp.exp(_allsum(acc) * INV_SQRT_MEM)
                outv[e, pl.ds(0, 16)] = ex * msg_v
                outv[e, pl.ds(16, 16)] = ex * te
                outv[e, pl.ds(32, 16)] = ex * unit
                exg = jnp.where(lane == j, ex, exg)
            exv[pl.ds(g * 16, 16)] = exg
            return gcarry

        lax.fori_loop(0, G, group, 0)
        pltpu.sync_copy(exv, ex_hbm.at[wid, pl.ds(base, C)])
        pltpu.sync_copy(outv, accum.at[dstv], add=True)
        return carry

    lax.fori_loop(0, NCH, chunk, 0)

    plsc.subcore_barrier()
    for j in range(NCP):
        r0 = s * RPT + j * CPR
        pltpu.sync_copy(accum.at[pl.ds(r0, CPR)], zb)
        pltpu.sync_copy(zb, out_hbm.at[c, pl.ds(r0, CPR)])


def _run_alpha(ktab, qe, src, dst, t, msg, wt, bt):
    f32 = jnp.float32
    mesh = plsc.VectorSubcoreMesh(core_axis_name="c", subcore_axis_name="s",
                                  num_cores=NC, num_subcores=NS)
    fn = pl.kernel(
        _alpha_body,
        out_type=[jax.ShapeDtypeStruct((NW, EPW), f32),
                  jax.ShapeDtypeStruct((NC, NPAD, SM_D), f32)],
        mesh=mesh,
        compiler_params=pltpu.CompilerParams(use_tc_tiling_on_sc=False),
        scratch_types=[
            pltpu.VMEM((C,), jnp.int32),
            pltpu.VMEM((C,), jnp.int32),
            pltpu.VMEM((C, K_D), f32),
            pltpu.VMEM((C, QE_D), f32),
            pltpu.VMEM((C, 16), f32),
            pltpu.VMEM((C + 16,), f32),
            pltpu.VMEM((C,), f32),
            pltpu.VMEM((C, SM_D), f32),
            pltpu.VMEM((1, 16), f32),
            pltpu.VMEM((1, 16), f32),
            pltpu.VMEM((CPR, SM_D), f32),
            pltpu.VMEM_SHARED((NPAD, SM_D), f32),
            pltpu.SemaphoreType.DMA,
        ],
    )
    return fn(ktab, qe, src, dst, t, msg, wt, bt)


def _vagg_body(vlo_hbm, vhi_hbm, ex_hbm, src_hbm, dst_hbm, out_hbm,
               srcv, dstv, vv, exv, outv, zb, accum, sem):
    c = lax.axis_index("c")
    s = lax.axis_index("s")

    zeros16 = jnp.zeros((16,), jnp.float32)

    def zrow(j, carry):
        for kk in range(V_D // 16):
            zb[j, pl.ds(kk * 16, 16)] = zeros16
        return carry

    lax.fori_loop(0, CPR, zrow, 0)
    for j in range(NCP):
        pltpu.sync_copy(zb, accum.at[pl.ds(s * RPT + j * CPR, CPR)])
    plsc.subcore_barrier()

    def chunk(i, carry):
        row = s * NR2 + i // NCH
        base = (i % NCH) * C
        pltpu.sync_copy(src_hbm.at[row, pl.ds(base, C)], srcv)
        pltpu.sync_copy(dst_hbm.at[row, pl.ds(base, C)], dstv)
        pltpu.sync_copy(ex_hbm.at[row, pl.ds(base, C)], exv.at[pl.ds(0, C)])

        @pl.when(c == 0)
        def _():
            pltpu.async_copy(vlo_hbm.at[srcv], vv, sem).wait()

        @pl.when(c == 1)
        def _():
            pltpu.async_copy(vhi_hbm.at[srcv], vv, sem).wait()

        def edge(e, ecarry):
            ex = exv[pl.ds(e, 16)][0]
            for r in range(V_D // 16):
                outv[e, pl.ds(16 * r, 16)] = ex * vv[e, pl.ds(16 * r, 16)]
            return ecarry

        lax.fori_loop(0, C, edge, 0, unroll=2)
        pltpu.sync_copy(outv, accum.at[dstv], add=True)
        return carry

    lax.fori_loop(0, NR2 * NCH, chunk, 0)

    plsc.subcore_barrier()
    for j in range(NCP):
        r0 = s * RPT + j * CPR
        pltpu.sync_copy(accum.at[pl.ds(r0, CPR)], zb)
        pltpu.sync_copy(zb, out_hbm.at[c, pl.ds(r0, CPR)])


def _run_vagg(vlo, vhi, ex, src, dst):
    f32 = jnp.float32
    mesh = plsc.VectorSubcoreMesh(core_axis_name="c", subcore_axis_name="s",
                                  num_cores=NC, num_subcores=NS)
    fn = pl.kernel(
        _vagg_body,
        out_type=jax.ShapeDtypeStruct((NC, NPAD, V_D), f32),
        mesh=mesh,
        compiler_params=pltpu.CompilerParams(use_tc_tiling_on_sc=False),
        scratch_types=[
            pltpu.VMEM((C,), jnp.int32),
            pltpu.VMEM((C,), jnp.int32),
            pltpu.VMEM((C, V_D), f32),
            pltpu.VMEM((C + 16,), f32),
            pltpu.VMEM((C, V_D), f32),
            pltpu.VMEM((CPR, V_D), f32),
            pltpu.VMEM_SHARED((NPAD, V_D), f32),
            pltpu.SemaphoreType.DMA,
        ],
    )
    return fn(vlo, vhi, ex, src, dst)


def _post_body(sm_ref, vagg_ref, enc_ref, We_ref, be_row_ref, Wa_ref,
               ba_row_ref, out_ref):
    f32 = jnp.float32
    S = sm_ref[0] + sm_ref[1]
    Sm = S[:, 0:16]
    St = S[:, 16:32]
    Sd = S[:, 32:33]
    Sv = jnp.concatenate([vagg_ref[0], vagg_ref[1]], axis=1)
    We = We_ref[...]
    dotT = lambda a, w: lax.dot_general(a, w, (((1,), (1,)), ((), ())),
                                        preferred_element_type=f32)
    num = Sv + dotT(Sm, We[:, 0:16]) + dotT(St, We[:, 16:32]) \
        + Sd * be_row_ref[...]
    conv = num / (Sd + 1e-16)
    enc = enc_ref[...]
    Wa = Wa_ref[...]
    lin = dotT(enc, Wa) - jnp.dot(enc, Wa, preferred_element_type=f32) \
        - GAMMA * enc
    h = jnp.tanh(lin + conv + ba_row_ref[...])
    out_ref[...] = jnp.tanh(enc + EPSILON * h)


def _run_post(sm, vagg, enc, We, be_row, Wa, ba_row):
    f32 = jnp.float32
    return pl.pallas_call(
        _post_body,
        grid=(GRID,),
        in_specs=[pl.BlockSpec((NC, BR, SM_D), lambda i: (0, i, 0)),
                  pl.BlockSpec((NC, BR, V_D), lambda i: (0, i, 0)),
                  pl.BlockSpec((BR, MEM), lambda i: (i, 0)),
                  pl.BlockSpec((MEM, 32), lambda i: (0, 0)),
                  pl.BlockSpec((1, MEM), lambda i: (0, 0)),
                  pl.BlockSpec((MEM, MEM), lambda i: (0, 0)),
                  pl.BlockSpec((1, MEM), lambda i: (0, 0))],
        out_specs=pl.BlockSpec((BR, MEM), lambda i: (i, 0)),
        out_shape=jax.ShapeDtypeStruct((N, MEM), f32),
    )(sm, vagg, enc, We, be_row, Wa, ba_row)


def kernel(x, last_update, edge_index, t, msg, W_time, b_time, W_enc, b_enc,
           Wq, bq, Wk, bk, Wv, bv, We, be, W_asym, b_asym):
    lu2 = last_update.reshape(N, 1)
    row = lambda b: b.reshape(1, MEM)
    ktab, vlo, vhi, qe, enc = _run_pre(x, lu2, W_enc, row(b_enc), Wq, row(bq),
                                       Wk, row(bk), Wv, row(bv), We,
                                       be.reshape(MEM, 1))
    src = edge_index[0].reshape(NW, EPW)
    dst = edge_index[1].reshape(NW, EPW)
    ex, sm = _run_alpha(ktab, qe, src, dst, t.reshape(NW, EPW), msg,
                        W_time.reshape(1, 16), b_time.reshape(1, 16))
    vagg = _run_vagg(vlo, vhi, ex, src, dst)
    return _run_post(sm, vagg, enc, We, row(be), W_asym, row(b_asym))


# speedup vs baseline: 5.6930x; 1.4184x over previous
import functools

import jax
import jax.numpy as jnp
from jax import lax
from jax.experimental import pallas as pl
from jax.experimental.pallas import tpu as pltpu
from jax.experimental.pallas import tpu_sc as plsc

N = 10000
E = 320000
MEM = 128
GAMMA = 0.1
EPSILON = 1.0
INV_SQRT_MEM = 1.0 / (128.0 ** 0.5)

K_D = 144
QE_D = 176
SM_D = 48
V_D = 64

NC = 2
NS = 16
NW = NC * NS
EPW = E // NW
C = 80
NCH = EPW // C
G = C // 16
NPAD = 10240
RPT = NPAD // NS
CPR = 128
NCP = RPT // CPR
EPT2 = E // NS
NR2 = EPT2 // EPW

BR = 200
GRID = N // BR

TWO_PI = 6.283185307179586
INV_2PI = 1.0 / TWO_PI
COS_COEF = (0.9999999999193508, -19.739208758208584, 64.93939011340913,
            -85.45668538180254, 60.24246470872289, -26.406761080377983,
            7.806608463960106, -1.4609479689305238)


def _cos_poly(z):
    u = z * INV_2PI
    n = u.astype(jnp.int32).astype(jnp.float32)
    f = u - n
    f = jnp.where(f > 0.5, f - 1.0, f)
    f = jnp.where(f < -0.5, f + 1.0, f)
    y = f * f
    acc = jnp.full_like(y, COS_COEF[7])
    for coef in COS_COEF[6::-1]:
        acc = acc * y + coef
    return acc


def _pre_body(x_ref, lu_ref, W_enc_ref, b_enc_ref, Wq_ref, bq_ref, Wk_ref,
              bk_ref, Wv_ref, bv_ref, We_ref, be_col_ref,
              k_ref, vlo_ref, vhi_ref, qe_ref, enc_ref):
    f32 = jnp.float32
    x = x_ref[...]
    dotT = lambda a, w: lax.dot_general(a, w, (((1,), (1,)), ((), ())),
                                        preferred_element_type=f32)
    enc = dotT(x, W_enc_ref[...]) + b_enc_ref[...]
    q = dotT(enc, Wq_ref[...]) + bq_ref[...]
    k = dotT(enc, Wk_ref[...]) + bk_ref[...]
    v = dotT(enc, Wv_ref[...]) + bv_ref[...]
    em = jnp.dot(q, We_ref[...], preferred_element_type=f32)
    qb = jnp.dot(q, be_col_ref[...], preferred_element_type=f32)
    pad = jnp.zeros((BR, 15), dtype=f32)
    k_ref[...] = jnp.concatenate([k, lu_ref[...], pad], axis=1)
    vlo_ref[...] = v[:, 0:V_D]
    vhi_ref[...] = v[:, V_D:MEM]
    qe_ref[...] = jnp.concatenate([q, em, qb, pad], axis=1)
    enc_ref[...] = enc


def _run_pre(x, lu2, W_enc, b_enc, Wq, bq, Wk, bk, Wv, bv, We, be_col):
    f32 = jnp.float32
    row = lambda d: pl.BlockSpec((BR, d), lambda i: (i, 0))
    full = lambda a, b: pl.BlockSpec((a, b), lambda i: (0, 0))
    return pl.pallas_call(
        _pre_body,
        grid=(GRID,),
        in_specs=[row(MEM), row(1), full(MEM, MEM), full(1, MEM),
                  full(MEM, MEM), full(1, MEM), full(MEM, MEM), full(1, MEM),
                  full(MEM, MEM), full(1, MEM), full(MEM, 32), full(MEM, 1)],
        out_specs=[row(K_D), row(V_D), row(V_D), row(QE_D), row(MEM)],
        out_shape=[jax.ShapeDtypeStruct((N, K_D), f32),
                   jax.ShapeDtypeStruct((N, V_D), f32),
                   jax.ShapeDtypeStruct((N, V_D), f32),
                   jax.ShapeDtypeStruct((N, QE_D), f32),
                   jax.ShapeDtypeStruct((N, MEM), f32)],
    )(x, lu2, W_enc, b_enc, Wq, bq, Wk, bk, Wv, bv, We, be_col)


def _alpha_body(k_hbm, qe_hbm, src_hbm, dst_hbm, t_hbm, msg_hbm, wt_hbm,
                bt_hbm, ex_hbm, out_hbm,
                srcvA, dstvA, kvvA, qevA, msgvA, tvA, outvA,
                srcvB, dstvB, kvvB, qevB, msgvB, tvB, outvB,
                exall, wtv, btv, zb, accum, semA, semB, semL):
    c = lax.axis_index("c")
    s = lax.axis_index("s")
    wid = s * NC + c

    pltpu.sync_copy(wt_hbm, wtv)
    pltpu.sync_copy(bt_hbm, btv)
    wt = wtv[0, pl.ds(0, 16)]
    bt = btv[0, pl.ds(0, 16)]
    lane = lax.iota(jnp.int32, 16)
    unit = jnp.where(lane == 0, 1.0, 0.0).astype(jnp.float32)
    _dn = lax.GatherDimensionNumbers(offset_dims=(), collapsed_slice_dims=(0,),
                                     start_index_map=(0,))
    _perms = [(lane ^ m)[:, None] for m in (8, 4, 2, 1)]

    def _allsum(a):
        for p in _perms:
            a = a + lax.gather(a, p, _dn, slice_sizes=(1,),
                               mode=lax.GatherScatterMode.PROMISE_IN_BOUNDS)
        return a

    zeros16 = jnp.zeros((16,), jnp.float32)

    def zrow(j, carry):
        for kk in range(SM_D // 16):
            zb[j, pl.ds(kk * 16, 16)] = zeros16
        return carry

    lax.fori_loop(0, CPR, zrow, 0)
    for j in range(NCP):
        pltpu.sync_copy(zb, accum.at[pl.ds(s * RPT + j * CPR, CPR)])
    plsc.subcore_barrier()

    def lin_sync(ci, srcv, dstv, tv, msgv):
        base = ci * C
        pltpu.sync_copy(src_hbm.at[wid, pl.ds(base, C)], srcv)
        pltpu.sync_copy(dst_hbm.at[wid, pl.ds(base, C)], dstv)
        pltpu.sync_copy(t_hbm.at[wid, pl.ds(base, C)], tv.at[pl.ds(0, C)])
        pltpu.sync_copy(msg_hbm.at[pl.ds(wid * EPW + base, C)], msgv)

    def lin_async(ci, srcv, dstv, tv, msgv):
        base = ci * C
        return [
            pltpu.async_copy(src_hbm.at[wid, pl.ds(base, C)], srcv, semL),
            pltpu.async_copy(dst_hbm.at[wid, pl.ds(base, C)], dstv, semL),
            pltpu.async_copy(t_hbm.at[wid, pl.ds(base, C)], tv.at[pl.ds(0, C)], semL),
            pltpu.async_copy(msg_hbm.at[pl.ds(wid * EPW + base, C)], msgv, semL),
        ]

    def gat_issue(srcv, dstv, kvv, qev, sem):
        pltpu.async_copy(k_hbm.at[srcv], kvv, sem)
        pltpu.async_copy(qe_hbm.at[dstv], qev, sem)

    def gat_wait(srcv, dstv, kvv, qev, sem):
        pltpu.make_async_copy(k_hbm.at[srcv], kvv, sem).wait()
        pltpu.make_async_copy(qe_hbm.at[dstv], qev, sem).wait()

    def compute(ci, kvv, qev, msgv, tv, outv, dstv):
        def group(g, gcarry):
            exg = zeros16
            for j in range(16):
                e = g * 16 + j
                acc = qev[e, pl.ds(0, 16)] * kvv[e, pl.ds(0, 16)]
                for r in range(1, 8):
                    acc = acc + qev[e, pl.ds(16 * r, 16)] * kvv[e, pl.ds(16 * r, 16)]
                msg_v = msgv[e, pl.ds(0, 16)]
                lu = kvv[e, pl.ds(128, 16)][0]
                t_e = tv[pl.ds(e, 16)][0]
                rel = jnp.abs(lu - t_e)
                te = _cos_poly(rel * wt + bt)
                acc = acc + qev[e, pl.ds(128, 16)] * msg_v
                acc = acc + qev[e, pl.ds(144, 16)] * te
                acc = acc + qev[e, pl.ds(160, 16)]
                ex = jnp.exp(_allsum(acc) * INV_SQRT_MEM)
                outv[e, pl.ds(0, 16)] = ex * msg_v
                outv[e, pl.ds(16, 16)] = ex * te
                outv[e, pl.ds(32, 16)] = ex * unit
                exg = jnp.where(lane == j, ex, exg)
            exall[pl.ds(ci * C + g * 16, 16)] = exg
            return gcarry

        lax.fori_loop(0, G, group, 0)
        pltpu.sync_copy(outv, accum.at[dstv], add=True)

    lin_sync(0, srcvA, dstvA, tvA, msgvA)
    gat_issue(srcvA, dstvA, kvvA, qevA, semA)
    lin_sync(1, srcvB, dstvB, tvB, msgvB)

    def pair(p, carry):
        a = 2 * p
        gat_issue(srcvB, dstvB, kvvB, qevB, semB)
        gat_wait(srcvA, dstvA, kvvA, qevA, semA)
        compute(a, kvvA, qevA, msgvA, tvA, outvA, dstvA)
        nxtA = jnp.minimum(a + 2, NCH - 1)
        la = lin_async(nxtA, srcvA, dstvA, tvA, msgvA)
        gat_wait(srcvB, dstvB, kvvB, qevB, semB)
        compute(a + 1, kvvB, qevB, msgvB, tvB, outvB, dstvB)
        for d in la:
            d.wait()
        gat_issue(srcvA, dstvA, kvvA, qevA, semA)
        lin_sync(jnp.minimum(a + 3, NCH - 1), srcvB, dstvB, tvB, msgvB)
        return carry

    lax.fori_loop(0, NCH // 2, pair, 0)
    gat_wait(srcvA, dstvA, kvvA, qevA, semA)
    compute(NCH - 1, kvvA, qevA, msgvA, tvA, outvA, dstvA)

    pltpu.sync_copy(exall, ex_hbm.at[wid])

    plsc.subcore_barrier()
    for j in range(NCP):
        r0 = s * RPT + j * CPR
        pltpu.sync_copy(accum.at[pl.ds(r0, CPR)], zb)
        pltpu.sync_copy(zb, out_hbm.at[c, pl.ds(r0, CPR)])


def _run_alpha(ktab, qe, src, dst, t, msg, wt, bt):
    f32 = jnp.float32
    mesh = plsc.VectorSubcoreMesh(core_axis_name="c", subcore_axis_name="s",
                                  num_cores=NC, num_subcores=NS)
    fn = pl.kernel(
        _alpha_body,
        out_type=[jax.ShapeDtypeStruct((NW, EPW), f32),
                  jax.ShapeDtypeStruct((NC, NPAD, SM_D), f32)],
        mesh=mesh,
        compiler_params=pltpu.CompilerParams(use_tc_tiling_on_sc=False),
        scratch_types=(
            2 * [
                pltpu.VMEM((C,), jnp.int32),
                pltpu.VMEM((C,), jnp.int32),
                pltpu.VMEM((C, K_D), f32),
                pltpu.VMEM((C, QE_D), f32),
                pltpu.VMEM((C, 16), f32),
                pltpu.VMEM((C + 16,), f32),
                pltpu.VMEM((C, SM_D), f32),
            ]
            + [
                pltpu.VMEM((EPW,), f32),
                pltpu.VMEM((1, 16), f32),
                pltpu.VMEM((1, 16), f32),
                pltpu.VMEM((CPR, SM_D), f32),
                pltpu.VMEM_SHARED((NPAD, SM_D), f32),
                pltpu.SemaphoreType.DMA,
                pltpu.SemaphoreType.DMA,
                pltpu.SemaphoreType.DMA,
            ]
        ),
    )
    return fn(ktab, qe, src, dst, t, msg, wt, bt)


def _vagg_body(vlo_hbm, vhi_hbm, ex_hbm, src_hbm, dst_hbm, out_hbm,
               srcvA, dstvA, vvA, exvA, outvA,
               srcvB, dstvB, vvB, exvB, outvB,
               zb, accum, semA, semB, semL):
    c = lax.axis_index("c")
    s = lax.axis_index("s")

    zeros16 = jnp.zeros((16,), jnp.float32)

    def zrow(j, carry):
        for kk in range(V_D // 16):
            zb[j, pl.ds(kk * 16, 16)] = zeros16
        return carry

    lax.fori_loop(0, CPR, zrow, 0)
    for j in range(NCP):
        pltpu.sync_copy(zb, accum.at[pl.ds(s * RPT + j * CPR, CPR)])
    plsc.subcore_barrier()

    NCH2 = NR2 * NCH

    def lin_sync(ci, srcv, dstv, exv):
        row = s * NR2 + ci // NCH
        base = (ci % NCH) * C
        pltpu.sync_copy(src_hbm.at[row, pl.ds(base, C)], srcv)
        pltpu.sync_copy(dst_hbm.at[row, pl.ds(base, C)], dstv)
        pltpu.sync_copy(ex_hbm.at[row, pl.ds(base, C)], exv.at[pl.ds(0, C)])

    def lin_async(ci, srcv, dstv, exv):
        row = s * NR2 + ci // NCH
        base = (ci % NCH) * C
        return [
            pltpu.async_copy(src_hbm.at[row, pl.ds(base, C)], srcv, semL),
            pltpu.async_copy(dst_hbm.at[row, pl.ds(base, C)], dstv, semL),
            pltpu.async_copy(ex_hbm.at[row, pl.ds(base, C)], exv.at[pl.ds(0, C)], semL),
        ]

    def gat_issue(srcv, vv, sem):
        @pl.when(c == 0)
        def _():
            pltpu.async_copy(vlo_hbm.at[srcv], vv, sem)

        @pl.when(c == 1)
        def _():
            pltpu.async_copy(vhi_hbm.at[srcv], vv, sem)

    def gat_wait(srcv, vv, sem):
        pltpu.make_async_copy(vlo_hbm.at[srcv], vv, sem).wait()

    def compute(vv, exv, outv, dstv):
        def edge(e, ecarry):
            ex = exv[pl.ds(e, 16)][0]
            for r in range(V_D // 16):
                outv[e, pl.ds(16 * r, 16)] = ex * vv[e, pl.ds(16 * r, 16)]
            return ecarry

        lax.fori_loop(0, C, edge, 0, unroll=4)
        pltpu.sync_copy(outv, accum.at[dstv], add=True)

    lin_sync(0, srcvA, dstvA, exvA)
    gat_issue(srcvA, vvA, semA)
    lin_sync(1, srcvB, dstvB, exvB)

    def pair(p, carry):
        a = 2 * p
        gat_issue(srcvB, vvB, semB)
        gat_wait(srcvA, vvA, semA)
        compute(vvA, exvA, outvA, dstvA)
        la = lin_async(jnp.minimum(a + 2, NCH2 - 1), srcvA, dstvA, exvA)
        gat_wait(srcvB, vvB, semB)
        compute(vvB, exvB, outvB, dstvB)
        for d in la:
            d.wait()
        gat_issue(srcvA, vvA, semA)
        lin_sync(jnp.minimum(a + 3, NCH2 - 1), srcvB, dstvB, exvB)
        return carry

    lax.fori_loop(0, NCH2 // 2, pair, 0)
    gat_wait(srcvA, vvA, semA)

    plsc.subcore_barrier()
    for j in range(NCP):
        r0 = s * RPT + j * CPR
        pltpu.sync_copy(accum.at[pl.ds(r0, CPR)], zb)
        pltpu.sync_copy(zb, out_hbm.at[c, pl.ds(r0, CPR)])


def _run_vagg(vlo, vhi, ex, src, dst):
    f32 = jnp.float32
    mesh = plsc.VectorSubcoreMesh(core_axis_name="c", subcore_axis_name="s",
                                  num_cores=NC, num_subcores=NS)
    fn = pl.kernel(
        _vagg_body,
        out_type=jax.ShapeDtypeStruct((NC, NPAD, V_D), f32),
        mesh=mesh,
        compiler_params=pltpu.CompilerParams(use_tc_tiling_on_sc=False),
        scratch_types=(
            2 * [
                pltpu.VMEM((C,), jnp.int32),
                pltpu.VMEM((C,), jnp.int32),
                pltpu.VMEM((C, V_D), f32),
                pltpu.VMEM((C + 16,), f32),
                pltpu.VMEM((C, V_D), f32),
            ]
            + [
                pltpu.VMEM((CPR, V_D), f32),
                pltpu.VMEM_SHARED((NPAD, V_D), f32),
                pltpu.SemaphoreType.DMA,
                pltpu.SemaphoreType.DMA,
                pltpu.SemaphoreType.DMA,
            ]
        ),
    )
    return fn(vlo, vhi, ex, src, dst)


def _post_body(sm_ref, vagg_ref, enc_ref, We_ref, be_row_ref, Wa_ref,
               ba_row_ref, out_ref):
    f32 = jnp.float32
    S = sm_ref[0] + sm_ref[1]
    Sm = S[:, 0:16]
    St = S[:, 16:32]
    Sd = S[:, 32:33]
    Sv = jnp.concatenate([vagg_ref[0], vagg_ref[1]], axis=1)
    We = We_ref[...]
    dotT = lambda a, w: lax.dot_general(a, w, (((1,), (1,)), ((), ())),
                                        preferred_element_type=f32)
    num = Sv + dotT(Sm, We[:, 0:16]) + dotT(St, We[:, 16:32]) \
        + Sd * be_row_ref[...]
    conv = num / (Sd + 1e-16)
    enc = enc_ref[...]
    Wa = Wa_ref[...]
    lin = dotT(enc, Wa) - jnp.dot(enc, Wa, preferred_element_type=f32) \
        - GAMMA * enc
    h = jnp.tanh(lin + conv + ba_row_ref[...])
    out_ref[...] = jnp.tanh(enc + EPSILON * h)


def _run_post(sm, vagg, enc, We, be_row, Wa, ba_row):
    f32 = jnp.float32
    return pl.pallas_call(
        _post_body,
        grid=(GRID,),
        in_specs=[pl.BlockSpec((NC, BR, SM_D), lambda i: (0, i, 0)),
                  pl.BlockSpec((NC, BR, V_D), lambda i: (0, i, 0)),
                  pl.BlockSpec((BR, MEM), lambda i: (i, 0)),
                  pl.BlockSpec((MEM, 32), lambda i: (0, 0)),
                  pl.BlockSpec((1, MEM), lambda i: (0, 0)),
                  pl.BlockSpec((MEM, MEM), lambda i: (0, 0)),
                  pl.BlockSpec((1, MEM), lambda i: (0, 0))],
        out_specs=pl.BlockSpec((BR, MEM), lambda i: (i, 0)),
        out_shape=jax.ShapeDtypeStruct((N, MEM), f32),
    )(sm, vagg, enc, We, be_row, Wa, ba_row)


def kernel(x, last_update, edge_index, t, msg, W_time, b_time, W_enc, b_enc,
           Wq, bq, Wk, bk, Wv, bv, We, be, W_asym, b_asym):
    lu2 = last_update.reshape(N, 1)
    row = lambda b: b.reshape(1, MEM)
    ktab, vlo, vhi, qe, enc = _run_pre(x, lu2, W_enc, row(b_enc), Wq, row(bq),
                                       Wk, row(bk), Wv, row(bv), We,
                                       be.reshape(MEM, 1))
    src = edge_index[0].reshape(NW, EPW)
    dst = edge_index[1].reshape(NW, EPW)
    ex, sm = _run_alpha(ktab, qe, src, dst, t.reshape(NW, EPW), msg,
                        W_time.reshape(1, 16), b_time.reshape(1, 16))
    vagg = _run_vagg(vlo, vhi, ex, src, dst)
    return _run_post(sm, vagg, enc, We, row(be), W_asym, row(b_asym))


# speedup vs baseline: 6.2179x; 1.0922x over previous
import jax
import jax.numpy as jnp
from jax import lax
from jax.experimental import pallas as pl
from jax.experimental.pallas import tpu as pltpu
from jax.experimental.pallas import tpu_sc as plsc

N = 10000
E = 320000
MEM = 128
GAMMA = 0.1
EPSILON = 1.0
INV_SQRT_MEM = 1.0 / (128.0 ** 0.5)

K_D = 144
QE_D = 176
SM_D = 48
V_D = 64

NC = 2
NS = 16
NW = NC * NS
N2 = 10240
E2 = 327680
EPW = E2 // NW
C = 80
NCH = EPW // C
G = C // 16
RPT = N2 // NS
CPR = 128
NCP = RPT // CPR
NCH2 = 2 * NCH

BRP = 256
GRIDP = N2 // BRP
BR = 200
GRID = N // BR

TWO_PI = 6.283185307179586
INV_2PI = 1.0 / TWO_PI
COS_COEF = (0.9999999999193508, -19.739208758208584, 64.93939011340913,
            -85.45668538180254, 60.24246470872289, -26.406761080377983,
            7.806608463960106, -1.4609479689305238)


def _cos_poly(z):
    u = z * INV_2PI
    n = u.astype(jnp.int32).astype(jnp.float32)
    f = u - n
    f = jnp.where(f > 0.5, f - 1.0, f)
    f = jnp.where(f < -0.5, f + 1.0, f)
    y = f * f
    acc = jnp.full_like(y, COS_COEF[7])
    for coef in COS_COEF[6::-1]:
        acc = acc * y + coef
    return acc


def _pre_body(x_ref, lu_ref, W_enc_ref, b_enc_ref, Wq_ref, bq_ref, Wk_ref,
              bk_ref, Wv_ref, bv_ref, We_ref, be_col_ref,
              k_ref, vlo_ref, vhi_ref, qe_ref, enc_ref):
    f32 = jnp.float32
    x = x_ref[...]
    dotT = lambda a, w: lax.dot_general(a, w, (((1,), (1,)), ((), ())),
                                        preferred_element_type=f32)
    enc = dotT(x, W_enc_ref[...]) + b_enc_ref[...]
    q = dotT(enc, Wq_ref[...]) + bq_ref[...]
    k = dotT(enc, Wk_ref[...]) + bk_ref[...]
    v = dotT(enc, Wv_ref[...]) + bv_ref[...]
    em = jnp.dot(q, We_ref[...], preferred_element_type=f32)
    qb = jnp.dot(q, be_col_ref[...], preferred_element_type=f32)
    pad = jnp.zeros((BRP, 15), dtype=f32)
    k_ref[...] = jnp.concatenate([k, lu_ref[...], pad], axis=1)
    vlo_ref[...] = v[:, 0:V_D]
    vhi_ref[...] = v[:, V_D:MEM]
    qe_ref[...] = jnp.concatenate([q, em, qb, pad], axis=1)
    enc_ref[...] = enc


def _run_pre(x, lu2, W_enc, b_enc, Wq, bq, Wk, bk, Wv, bv, We, be_col):
    f32 = jnp.float32
    row = lambda d: pl.BlockSpec((BRP, d), lambda i: (i, 0))
    full = lambda a, b: pl.BlockSpec((a, b), lambda i: (0, 0))
    return pl.pallas_call(
        _pre_body,
        grid=(GRIDP,),
        in_specs=[row(MEM), row(1), full(MEM, MEM), full(1, MEM),
                  full(MEM, MEM), full(1, MEM), full(MEM, MEM), full(1, MEM),
                  full(MEM, MEM), full(1, MEM), full(MEM, 32), full(MEM, 1)],
        out_specs=[row(K_D), row(V_D), row(V_D), row(QE_D), row(MEM)],
        out_shape=[jax.ShapeDtypeStruct((N2, K_D), f32),
                   jax.ShapeDtypeStruct((N2, V_D), f32),
                   jax.ShapeDtypeStruct((N2, V_D), f32),
                   jax.ShapeDtypeStruct((N2, QE_D), f32),
                   jax.ShapeDtypeStruct((N2, MEM), f32)],
    )(x, lu2, W_enc, b_enc, Wq, bq, Wk, bk, Wv, bv, We, be_col)


def _alpha_body(k_hbm, qe_hbm, src_hbm, dst_hbm, t_hbm, msg_hbm, wt_hbm,
                bt_hbm, ex_hbm, out_hbm,
                srcv0, dstv0, tv0, msgv0, srcv1, dstv1, tv1, msgv1,
                srcv2, dstv2, tv2, msgv2, srcv3, dstv3, tv3, msgv3,
                kvv0, qev0, kvv1, qev1, outv0, outv1,
                exall, wtv, btv, zb, accum,
                semL0, semL1, semG0, semG1, semS0, semS1):
    c = lax.axis_index("c")
    s = lax.axis_index("s")
    wid = s * NC + c

    srcv = (srcv0, srcv1, srcv2, srcv3)
    dstv = (dstv0, dstv1, dstv2, dstv3)
    tv = (tv0, tv1, tv2, tv3)
    msgv = (msgv0, msgv1, msgv2, msgv3)
    kvv = (kvv0, kvv1)
    qev = (qev0, qev1)
    outv = (outv0, outv1)
    semL = (semL0, semL1)
    semG = (semG0, semG1)
    semS = (semS0, semS1)

    pltpu.sync_copy(wt_hbm, wtv)
    pltpu.sync_copy(bt_hbm, btv)
    wt = wtv[0, pl.ds(0, 16)]
    bt = btv[0, pl.ds(0, 16)]
    lane = lax.iota(jnp.int32, 16)
    unit = jnp.where(lane == 0, 1.0, 0.0).astype(jnp.float32)
    _dn = lax.GatherDimensionNumbers(offset_dims=(), collapsed_slice_dims=(0,),
                                     start_index_map=(0,))
    _perms = [(lane ^ m)[:, None] for m in (8, 4, 2, 1)]

    def _allsum(a):
        for p in _perms:
            a = a + lax.gather(a, p, _dn, slice_sizes=(1,),
                               mode=lax.GatherScatterMode.PROMISE_IN_BOUNDS)
        return a

    zeros16 = jnp.zeros((16,), jnp.float32)

    def zrow(j, carry):
        for kk in range(SM_D // 16):
            zb[j, pl.ds(kk * 16, 16)] = zeros16
        return carry

    lax.fori_loop(0, CPR, zrow, 0)
    for j in range(NCP):
        pltpu.sync_copy(zb, accum.at[pl.ds(s * RPT + j * CPR, CPR)])
    plsc.subcore_barrier()

    def lin_issue(ci, l4, l2):
        base = jnp.minimum(ci, NCH - 1) * C
        return [
            pltpu.async_copy(src_hbm.at[wid, pl.ds(base, C)], srcv[l4], semL[l2]),
            pltpu.async_copy(dst_hbm.at[wid, pl.ds(base, C)], dstv[l4], semL[l2]),
            pltpu.async_copy(t_hbm.at[wid, pl.ds(base, C)], tv[l4].at[pl.ds(0, C)], semL[l2]),
            pltpu.async_copy(msg_hbm.at[pl.ds(wid * EPW + base, C)], msgv[l4], semL[l2]),
        ]

    def lin_drain(l4, l2):
        pltpu.make_async_copy(src_hbm.at[wid, pl.ds(0, C)], srcv[l4], semL[l2]).wait()
        pltpu.make_async_copy(dst_hbm.at[wid, pl.ds(0, C)], dstv[l4], semL[l2]).wait()
        pltpu.make_async_copy(t_hbm.at[wid, pl.ds(0, C)], tv[l4].at[pl.ds(0, C)], semL[l2]).wait()
        pltpu.make_async_copy(msg_hbm.at[pl.ds(0, C)], msgv[l4], semL[l2]).wait()

    def gat_issue(l4, k2):
        pltpu.async_copy(k_hbm.at[srcv[l4]], kvv[k2], semG[k2])
        pltpu.async_copy(qe_hbm.at[dstv[l4]], qev[k2], semG[k2])

    def gat_drain(l4, k2):
        pltpu.make_async_copy(k_hbm.at[srcv[l4]], kvv[k2], semG[k2]).wait()
        pltpu.make_async_copy(qe_hbm.at[dstv[l4]], qev[k2], semG[k2]).wait()

    def sca_issue(l4, k2):
        pltpu.async_copy(outv[k2], accum.at[dstv[l4]], semS[k2], add=True)

    def sca_drain(l4, k2):
        pltpu.make_async_copy(outv[k2], accum.at[dstv[l4]], semS[k2]).wait()

    def compute(ci, l4, k2):
        kv = kvv[k2]
        qe = qev[k2]
        ov = outv[k2]
        mv = msgv[l4]
        tt = tv[l4]

        def group(g, gcarry):
            exg = zeros16
            for j in range(16):
                e = g * 16 + j
                acc = qe[e, pl.ds(0, 16)] * kv[e, pl.ds(0, 16)]
                for r in range(1, 8):
                    acc = acc + qe[e, pl.ds(16 * r, 16)] * kv[e, pl.ds(16 * r, 16)]
                msg_v = mv[e, pl.ds(0, 16)]
                lu = kv[e, pl.ds(128, 16)][0]
                t_e = tt[pl.ds(e, 16)][0]
                rel = jnp.abs(lu - t_e)
                te = _cos_poly(rel * wt + bt)
                acc = acc + qe[e, pl.ds(128, 16)] * msg_v
                acc = acc + qe[e, pl.ds(144, 16)] * te
                acc = acc + qe[e, pl.ds(160, 16)]
                ex = jnp.exp(_allsum(acc) * INV_SQRT_MEM)
                ov[e, pl.ds(0, 16)] = ex * msg_v
                ov[e, pl.ds(16, 16)] = ex * te
                ov[e, pl.ds(32, 16)] = ex * unit
                exg = jnp.where(lane == j, ex, exg)
            exall[pl.ds(ci * C + g * 16, 16)] = exg
            return gcarry

        lax.fori_loop(0, G, group, 0)

    lin_issue(0, 0, 0)
    lin_issue(1, 1, 1)
    lin_drain(0, 0)
    gat_issue(0, 0)

    def quad(q4, carry):
        i0 = 4 * q4
        for j in range(4):
            i = i0 + j
            l4 = j
            k2 = j % 2

            @pl.when(i >= 2)
            def _():
                sca_drain((j + 2) % 4, k2)

            la = lin_issue(i + 2, (j + 2) % 4, k2)
            lin_drain((j + 1) % 4, (j + 1) % 2)
            gat_issue((j + 1) % 4, (j + 1) % 2)
            gat_drain(l4, k2)
            compute(i, l4, k2)
            sca_issue(l4, k2)
            del la
        return carry

    lax.fori_loop(0, NCH // 4, quad, 0)
    sca_drain(2, 0)
    sca_drain(3, 1)
    lin_drain(1, 1)
    gat_drain(0, 0)

    pltpu.sync_copy(exall, ex_hbm.at[wid])

    plsc.subcore_barrier()
    for j in range(NCP):
        r0 = s * RPT + j * CPR
        pltpu.sync_copy(accum.at[pl.ds(r0, CPR)], zb)
        pltpu.sync_copy(zb, out_hbm.at[c, pl.ds(r0, CPR)])


def _run_alpha(ktab, qe, src, dst, t, msg, wt, bt):
    f32 = jnp.float32
    mesh = plsc.VectorSubcoreMesh(core_axis_name="c", subcore_axis_name="s",
                                  num_cores=NC, num_subcores=NS)
    fn = pl.kernel(
        _alpha_body,
        out_type=[jax.ShapeDtypeStruct((NW, EPW), f32),
                  jax.ShapeDtypeStruct((NC, N2, SM_D), f32)],
        mesh=mesh,
        compiler_params=pltpu.CompilerParams(use_tc_tiling_on_sc=False),
        scratch_types=(
            4 * [
                pltpu.VMEM((C,), jnp.int32),
                pltpu.VMEM((C,), jnp.int32),
                pltpu.VMEM((C + 16,), f32),
                pltpu.VMEM((C, 16), f32),
            ]
            + 2 * [
                pltpu.VMEM((C, K_D), f32),
                pltpu.VMEM((C, QE_D), f32),
            ]
            + 2 * [
                pltpu.VMEM((C, SM_D), f32),
            ]
            + [
                pltpu.VMEM((EPW,), f32),
                pltpu.VMEM((1, 16), f32),
                pltpu.VMEM((1, 16), f32),
                pltpu.VMEM((CPR, SM_D), f32),
                pltpu.VMEM_SHARED((N2, SM_D), f32),
            ]
            + 6 * [pltpu.SemaphoreType.DMA]
        ),
    )
    return fn(ktab, qe, src, dst, t, msg, wt, bt)


def _vagg_body(vlo_hbm, vhi_hbm, ex_hbm, src_hbm, dst_hbm, out_hbm,
               srcv0, dstv0, exv0, srcv1, dstv1, exv1,
               srcv2, dstv2, exv2, srcv3, dstv3, exv3,
               vv0, vv1, outv0, outv1, zb, accum,
               semL0, semL1, semG0, semG1, semS0, semS1):
    c = lax.axis_index("c")
    s = lax.axis_index("s")

    srcv = (srcv0, srcv1, srcv2, srcv3)
    dstv = (dstv0, dstv1, dstv2, dstv3)
    exv = (exv0, exv1, exv2, exv3)
    vv = (vv0, vv1)
    outv = (outv0, outv1)
    semL = (semL0, semL1)
    semG = (semG0, semG1)
    semS = (semS0, semS1)

    zeros16 = jnp.zeros((16,), jnp.float32)

    def zrow(j, carry):
        for kk in range(V_D // 16):
            zb[j, pl.ds(kk * 16, 16)] = zeros16
        return carry

    lax.fori_loop(0, CPR, zrow, 0)
    for j in range(NCP):
        pltpu.sync_copy(zb, accum.at[pl.ds(s * RPT + j * CPR, CPR)])
    plsc.subcore_barrier()

    def lin_issue(ci, l4, l2):
        cc = jnp.minimum(ci, NCH2 - 1)
        row = s * 2 + cc // NCH
        base = (cc % NCH) * C
        return [
            pltpu.async_copy(src_hbm.at[row, pl.ds(base, C)], srcv[l4], semL[l2]),
            pltpu.async_copy(dst_hbm.at[row, pl.ds(base, C)], dstv[l4], semL[l2]),
            pltpu.async_copy(ex_hbm.at[row, pl.ds(base, C)], exv[l4].at[pl.ds(0, C)], semL[l2]),
        ]

    def lin_drain(l4, l2):
        pltpu.make_async_copy(src_hbm.at[0, pl.ds(0, C)], srcv[l4], semL[l2]).wait()
        pltpu.make_async_copy(dst_hbm.at[0, pl.ds(0, C)], dstv[l4], semL[l2]).wait()
        pltpu.make_async_copy(ex_hbm.at[0, pl.ds(0, C)], exv[l4].at[pl.ds(0, C)], semL[l2]).wait()

    def gat_issue(l4, k2):
        @pl.when(c == 0)
        def _():
            pltpu.async_copy(vlo_hbm.at[srcv[l4]], vv[k2], semG[k2])

        @pl.when(c == 1)
        def _():
            pltpu.async_copy(vhi_hbm.at[srcv[l4]], vv[k2], semG[k2])

    def gat_drain(l4, k2):
        pltpu.make_async_copy(vlo_hbm.at[srcv[l4]], vv[k2], semG[k2]).wait()

    def sca_issue(l4, k2):
        pltpu.async_copy(outv[k2], accum.at[dstv[l4]], semS[k2], add=True)

    def sca_drain(l4, k2):
        pltpu.make_async_copy(outv[k2], accum.at[dstv[l4]], semS[k2]).wait()

    def compute(l4, k2):
        ev = exv[l4]
        va = vv[k2]
        ov = outv[k2]

        def edge(e, ecarry):
            ex = ev[pl.ds(e, 16)][0]
            for r in range(V_D // 16):
                ov[e, pl.ds(16 * r, 16)] = ex * va[e, pl.ds(16 * r, 16)]
            return ecarry

        lax.fori_loop(0, C, edge, 0, unroll=4)

    lin_issue(0, 0, 0)
    lin_issue(1, 1, 1)
    lin_drain(0, 0)
    gat_issue(0, 0)

    def quad(q4, carry):
        i0 = 4 * q4
        for j in range(4):
            i = i0 + j
            l4 = j
            k2 = j % 2

            @pl.when(i >= 2)
            def _():
                sca_drain((j + 2) % 4, k2)

            la = lin_issue(i + 2, (j + 2) % 4, k2)
            lin_drain((j + 1) % 4, (j + 1) % 2)
            gat_issue((j + 1) % 4, (j + 1) % 2)
            gat_drain(l4, k2)
            compute(l4, k2)
            sca_issue(l4, k2)
            del la
        return carry

    lax.fori_loop(0, NCH2 // 4, quad, 0)
    sca_drain(2, 0)
    sca_drain(3, 1)
    lin_drain(1, 1)
    gat_drain(0, 0)

    plsc.subcore_barrier()
    for j in range(NCP):
        r0 = s * RPT + j * CPR
        pltpu.sync_copy(accum.at[pl.ds(r0, CPR)], zb)
        pltpu.sync_copy(zb, out_hbm.at[c, pl.ds(r0, CPR)])


def _run_vagg(vlo, vhi, ex, src, dst):
    f32 = jnp.float32
    mesh = plsc.VectorSubcoreMesh(core_axis_name="c", subcore_axis_name="s",
                                  num_cores=NC, num_subcores=NS)
    fn = pl.kernel(
        _vagg_body,
        out_type=jax.ShapeDtypeStruct((NC, N2, V_D), f32),
        mesh=mesh,
        compiler_params=pltpu.CompilerParams(use_tc_tiling_on_sc=False),
        scratch_types=(
            4 * [
                pltpu.VMEM((C,), jnp.int32),
                pltpu.VMEM((C,), jnp.int32),
                pltpu.VMEM((C + 16,), f32),
            ]
            + 2 * [
                pltpu.VMEM((C, V_D), f32),
            ]
            + 2 * [
                pltpu.VMEM((C, V_D), f32),
            ]
            + [
                pltpu.VMEM((CPR, V_D), f32),
                pltpu.VMEM_SHARED((N2, V_D), f32),
            ]
            + 6 * [pltpu.SemaphoreType.DMA]
        ),
    )
    return fn(vlo, vhi, ex, src, dst)


def _post_body(sm_ref, vagg_ref, enc_ref, We_ref, be_row_ref, Wa_ref,
               ba_row_ref, out_ref):
    f32 = jnp.float32
    S = sm_ref[0] + sm_ref[1]
    Sm = S[:, 0:16]
    St = S[:, 16:32]
    Sd = S[:, 32:33]
    Sv = jnp.concatenate([vagg_ref[0], vagg_ref[1]], axis=1)
    We = We_ref[...]
    dotT = lambda a, w: lax.dot_general(a, w, (((1,), (1,)), ((), ())),
                                        preferred_element_type=f32)
    num = Sv + dotT(Sm, We[:, 0:16]) + dotT(St, We[:, 16:32]) \
        + Sd * be_row_ref[...]
    conv = num / (Sd + 1e-16)
    enc = enc_ref[...]
    Wa = Wa_ref[...]
    lin = dotT(enc, Wa) - jnp.dot(enc, Wa, preferred_element_type=f32) \
        - GAMMA * enc
    h = jnp.tanh(lin + conv + ba_row_ref[...])
    out_ref[...] = jnp.tanh(enc + EPSILON * h)


def _run_post(sm, vagg, enc, We, be_row, Wa, ba_row):
    f32 = jnp.float32
    return pl.pallas_call(
        _post_body,
        grid=(GRID,),
        in_specs=[pl.BlockSpec((NC, BR, SM_D), lambda i: (0, i, 0)),
                  pl.BlockSpec((NC, BR, V_D), lambda i: (0, i, 0)),
                  pl.BlockSpec((BR, MEM), lambda i: (i, 0)),
                  pl.BlockSpec((MEM, 32), lambda i: (0, 0)),
                  pl.BlockSpec((1, MEM), lambda i: (0, 0)),
                  pl.BlockSpec((MEM, MEM), lambda i: (0, 0)),
                  pl.BlockSpec((1, MEM), lambda i: (0, 0))],
        out_specs=pl.BlockSpec((BR, MEM), lambda i: (i, 0)),
        out_shape=jax.ShapeDtypeStruct((N, MEM), f32),
    )(sm, vagg, enc, We, be_row, Wa, ba_row)


def kernel(x, last_update, edge_index, t, msg, W_time, b_time, W_enc, b_enc,
           Wq, bq, Wk, bk, Wv, bv, We, be, W_asym, b_asym):
    PN = N2 - N
    PE = E2 - E
    x2 = jnp.concatenate([x, jnp.zeros((PN, MEM), jnp.float32)], axis=0)
    lu2 = jnp.concatenate([last_update, jnp.zeros((PN,), jnp.float32)]
                          ).reshape(N2, 1)
    row = lambda b: b.reshape(1, MEM)
    ktab, vlo, vhi, qe, enc = _run_pre(x2, lu2, W_enc, row(b_enc), Wq, row(bq),
                                       Wk, row(bk), Wv, row(bv), We,
                                       be.reshape(MEM, 1))
    src = jnp.concatenate([edge_index[0], jnp.zeros((PE,), jnp.int32)]
                          ).reshape(NW, EPW)
    dst = jnp.concatenate([edge_index[1], jnp.full((PE,), N2 - 1, jnp.int32)]
                          ).reshape(NW, EPW)
    t2 = jnp.concatenate([t, jnp.zeros((PE,), jnp.float32)]).reshape(NW, EPW)
    msg2 = jnp.concatenate([msg, jnp.zeros((PE, 16), jnp.float32)], axis=0)
    ex, sm = _run_alpha(ktab, qe, src, dst, t2, msg2,
                        W_time.reshape(1, 16), b_time.reshape(1, 16))
    vagg = _run_vagg(vlo, vhi, ex, src, dst)
    return _run_post(sm, vagg, enc, We, row(be), W_asym, row(b_asym))


# speedup vs baseline: 7.1631x; 1.1520x over previous
import jax
import jax.numpy as jnp
from jax import lax
from jax.experimental import pallas as pl
from jax.experimental.pallas import tpu as pltpu
from jax.experimental.pallas import tpu_sc as plsc

N = 10000
E = 320000
MEM = 128
GAMMA = 0.1
EPSILON = 1.0
INV_SQRT_MEM = 1.0 / (128.0 ** 0.5)

K_D = 144
QE_D = 176
SM_D = 48
V_D = 64

NC = 2
NS = 16
NW = NC * NS
N2 = 10240
E2 = 327680
EPW = E2 // NW
C = 80
NCH = EPW // C
G = C // 16
RPT = N2 // NS
CPR = 128
NCP = RPT // CPR
NCH2 = 2 * NCH

BRP = 256
GRIDP = N2 // BRP
BR = 200
GRID = N // BR

TWO_PI = 6.283185307179586
INV_2PI = 1.0 / TWO_PI
COS_COEF = (0.9999999999193508, -19.739208758208584, 64.93939011340913,
            -85.45668538180254, 60.24246470872289, -26.406761080377983,
            7.806608463960106, -1.4609479689305238)


def _cos_poly(z):
    u = z * INV_2PI
    n = u.astype(jnp.int32).astype(jnp.float32)
    f = u - n
    f = jnp.where(f > 0.5, f - 1.0, f)
    f = jnp.where(f < -0.5, f + 1.0, f)
    y = f * f
    acc = jnp.full_like(y, COS_COEF[7])
    for coef in COS_COEF[6::-1]:
        acc = acc * y + coef
    return acc


def _pre_body(x_ref, lu_ref, W_enc_ref, b_enc_ref, Wq_ref, bq_ref, Wk_ref,
              bk_ref, Wv_ref, bv_ref, We_ref, be_col_ref,
              k_ref, vlo_ref, vhi_ref, qe_ref, enc_ref):
    f32 = jnp.float32
    x = x_ref[...]
    dotT = lambda a, w: lax.dot_general(a, w, (((1,), (1,)), ((), ())),
                                        preferred_element_type=f32)
    enc = dotT(x, W_enc_ref[...]) + b_enc_ref[...]
    q = dotT(enc, Wq_ref[...]) + bq_ref[...]
    k = dotT(enc, Wk_ref[...]) + bk_ref[...]
    v = dotT(enc, Wv_ref[...]) + bv_ref[...]
    em = jnp.dot(q, We_ref[...], preferred_element_type=f32)
    qb = jnp.dot(q, be_col_ref[...], preferred_element_type=f32)
    pad = jnp.zeros((BRP, 15), dtype=f32)
    k_ref[...] = jnp.concatenate([k, lu_ref[...], pad], axis=1)
    vlo_ref[...] = v[:, 0:V_D]
    vhi_ref[...] = v[:, V_D:MEM]
    qe_ref[...] = jnp.concatenate([q, em, qb, pad], axis=1)
    enc_ref[...] = enc


def _run_pre(x, lu2, W_enc, b_enc, Wq, bq, Wk, bk, Wv, bv, We, be_col):
    f32 = jnp.float32
    row = lambda d: pl.BlockSpec((BRP, d), lambda i: (i, 0))
    full = lambda a, b: pl.BlockSpec((a, b), lambda i: (0, 0))
    return pl.pallas_call(
        _pre_body,
        grid=(GRIDP,),
        in_specs=[row(MEM), row(1), full(MEM, MEM), full(1, MEM),
                  full(MEM, MEM), full(1, MEM), full(MEM, MEM), full(1, MEM),
                  full(MEM, MEM), full(1, MEM), full(MEM, 32), full(MEM, 1)],
        out_specs=[row(K_D), row(V_D), row(V_D), row(QE_D), row(MEM)],
        out_shape=[jax.ShapeDtypeStruct((N2, K_D), f32),
                   jax.ShapeDtypeStruct((N2, V_D), f32),
                   jax.ShapeDtypeStruct((N2, V_D), f32),
                   jax.ShapeDtypeStruct((N2, QE_D), f32),
                   jax.ShapeDtypeStruct((N2, MEM), f32)],
    )(x, lu2, W_enc, b_enc, Wq, bq, Wk, bk, Wv, bv, We, be_col)


def _alpha_body(k_hbm, qe_hbm, src_hbm, dst_hbm, t_hbm, msg_hbm, wt_hbm,
                bt_hbm, ex_hbm, out_hbm,
                srcv0, dstv0, tv0, msgv0, srcv1, dstv1, tv1, msgv1,
                srcv2, dstv2, tv2, msgv2, srcv3, dstv3, tv3, msgv3,
                kvv0, qev0, kvv1, qev1, outv0, outv1,
                exall, wtv, btv, zb, accum,
                semL0, semL1, semG0, semG1, semS0, semS1):
    c = lax.axis_index("c")
    s = lax.axis_index("s")
    wid = s * NC + c

    srcv = (srcv0, srcv1, srcv2, srcv3)
    dstv = (dstv0, dstv1, dstv2, dstv3)
    tv = (tv0, tv1, tv2, tv3)
    msgv = (msgv0, msgv1, msgv2, msgv3)
    kvv = (kvv0, kvv1)
    qev = (qev0, qev1)
    outv = (outv0, outv1)
    semL = (semL0, semL1)
    semG = (semG0, semG1)
    semS = (semS0, semS1)

    pltpu.sync_copy(wt_hbm, wtv)
    pltpu.sync_copy(bt_hbm, btv)
    wt = wtv[0, pl.ds(0, 16)]
    bt = btv[0, pl.ds(0, 16)]
    lane = lax.iota(jnp.int32, 16)
    unit = jnp.where(lane == 0, 1.0, 0.0).astype(jnp.float32)
    _dn = lax.GatherDimensionNumbers(offset_dims=(), collapsed_slice_dims=(0,),
                                     start_index_map=(0,))
    _perms = [(lane ^ m)[:, None] for m in (8, 4, 2, 1)]

    def _allsum(a):
        for p in _perms:
            a = a + lax.gather(a, p, _dn, slice_sizes=(1,),
                               mode=lax.GatherScatterMode.PROMISE_IN_BOUNDS)
        return a

    def _bcast(v, j):
        idx = jnp.full((16, 1), j, jnp.int32)
        return lax.gather(v, idx, _dn, slice_sizes=(1,),
                          mode=lax.GatherScatterMode.PROMISE_IN_BOUNDS)

    zeros16 = jnp.zeros((16,), jnp.float32)

    def zrow(j, carry):
        for kk in range(SM_D // 16):
            zb[j, pl.ds(kk * 16, 16)] = zeros16
        return carry

    lax.fori_loop(0, CPR, zrow, 0)
    for j in range(NCP):
        pltpu.sync_copy(zb, accum.at[pl.ds(s * RPT + j * CPR, CPR)])
    plsc.subcore_barrier()

    def lin_issue(ci, l4, l2):
        base = jnp.minimum(ci, NCH - 1) * C
        return [
            pltpu.async_copy(src_hbm.at[wid, pl.ds(base, C)], srcv[l4], semL[l2]),
            pltpu.async_copy(dst_hbm.at[wid, pl.ds(base, C)], dstv[l4], semL[l2]),
            pltpu.async_copy(t_hbm.at[wid, pl.ds(base, C)], tv[l4].at[pl.ds(0, C)], semL[l2]),
            pltpu.async_copy(msg_hbm.at[pl.ds(wid * EPW + base, C)], msgv[l4], semL[l2]),
        ]

    def lin_drain(l4, l2):
        pltpu.make_async_copy(src_hbm.at[wid, pl.ds(0, C)], srcv[l4], semL[l2]).wait()
        pltpu.make_async_copy(dst_hbm.at[wid, pl.ds(0, C)], dstv[l4], semL[l2]).wait()
        pltpu.make_async_copy(t_hbm.at[wid, pl.ds(0, C)], tv[l4].at[pl.ds(0, C)], semL[l2]).wait()
        pltpu.make_async_copy(msg_hbm.at[pl.ds(0, C)], msgv[l4], semL[l2]).wait()

    def gat_issue(l4, k2):
        pltpu.async_copy(k_hbm.at[srcv[l4]], kvv[k2], semG[k2])
        pltpu.async_copy(qe_hbm.at[dstv[l4]], qev[k2], semG[k2])

    def gat_drain(l4, k2):
        pltpu.make_async_copy(k_hbm.at[srcv[l4]], kvv[k2], semG[k2]).wait()
        pltpu.make_async_copy(qe_hbm.at[dstv[l4]], qev[k2], semG[k2]).wait()

    def sca_issue(l4, k2):
        pltpu.async_copy(outv[k2], accum.at[dstv[l4]], semS[k2], add=True)

    def sca_drain(l4, k2):
        pltpu.make_async_copy(outv[k2], accum.at[dstv[l4]], semS[k2]).wait()

    def compute(ci, l4, k2):
        kv = kvv[k2]
        qe = qev[k2]
        ov = outv[k2]
        mv = msgv[l4]
        tt = tv[l4]

        def group(g, gcarry):
            gi = g * 16
            t_g = tt[pl.ds(gi, 16)]
            exg = zeros16
            for j in range(16):
                e = gi + j
                acc = qe[e, pl.ds(0, 16)] * kv[e, pl.ds(0, 16)]
                for r in range(1, 8):
                    acc = acc + qe[e, pl.ds(16 * r, 16)] * kv[e, pl.ds(16 * r, 16)]
                msg_v = mv[e, pl.ds(0, 16)]
                rel = jnp.abs(_bcast(kv[e, pl.ds(128, 16)], 0) - _bcast(t_g, j))
                te = _cos_poly(rel * wt + bt)
                acc = acc + qe[e, pl.ds(128, 16)] * msg_v
                acc = acc + qe[e, pl.ds(144, 16)] * te
                acc = acc + qe[e, pl.ds(160, 16)]
                ex = jnp.exp(_allsum(acc) * INV_SQRT_MEM)
                ov[e, pl.ds(0, 16)] = ex * msg_v
                ov[e, pl.ds(16, 16)] = ex * te
                ov[e, pl.ds(32, 16)] = ex * unit
                exg = jnp.where(lane == j, ex, exg)
            exall[pl.ds(ci * C + g * 16, 16)] = exg
            return gcarry

        lax.fori_loop(0, G, group, 0)

    lin_issue(0, 0, 0)
    lin_issue(1, 1, 1)
    lin_drain(0, 0)
    gat_issue(0, 0)

    def quad(q4, carry):
        i0 = 4 * q4
        for j in range(4):
            i = i0 + j
            l4 = j
            k2 = j % 2

            @pl.when(i >= 2)
            def _():
                sca_drain((j + 2) % 4, k2)

            la = lin_issue(i + 2, (j + 2) % 4, k2)
            lin_drain((j + 1) % 4, (j + 1) % 2)
            gat_issue((j + 1) % 4, (j + 1) % 2)
            gat_drain(l4, k2)
            compute(i, l4, k2)
            sca_issue(l4, k2)
            del la
        return carry

    lax.fori_loop(0, NCH // 4, quad, 0)
    sca_drain(2, 0)
    sca_drain(3, 1)
    lin_drain(1, 1)
    gat_drain(0, 0)

    pltpu.sync_copy(exall, ex_hbm.at[wid])

    plsc.subcore_barrier()
    for j in range(NCP):
        r0 = s * RPT + j * CPR
        pltpu.sync_copy(accum.at[pl.ds(r0, CPR)], zb)
        pltpu.sync_copy(zb, out_hbm.at[c, pl.ds(r0, CPR)])


def _run_alpha(ktab, qe, src, dst, t, msg, wt, bt):
    f32 = jnp.float32
    mesh = plsc.VectorSubcoreMesh(core_axis_name="c", subcore_axis_name="s",
                                  num_cores=NC, num_subcores=NS)
    fn = pl.kernel(
        _alpha_body,
        out_type=[jax.ShapeDtypeStruct((NW, EPW), f32),
                  jax.ShapeDtypeStruct((NC, N2, SM_D), f32)],
        mesh=mesh,
        compiler_params=pltpu.CompilerParams(use_tc_tiling_on_sc=False),
        scratch_types=(
            4 * [
                pltpu.VMEM((C,), jnp.int32),
                pltpu.VMEM((C,), jnp.int32),
                pltpu.VMEM((C + 16,), f32),
                pltpu.VMEM((C, 16), f32),
            ]
            + 2 * [
                pltpu.VMEM((C, K_D), f32),
                pltpu.VMEM((C, QE_D), f32),
            ]
            + 2 * [
                pltpu.VMEM((C, SM_D), f32),
            ]
            + [
                pltpu.VMEM((EPW,), f32),
                pltpu.VMEM((1, 16), f32),
                pltpu.VMEM((1, 16), f32),
                pltpu.VMEM((CPR, SM_D), f32),
                pltpu.VMEM_SHARED((N2, SM_D), f32),
            ]
            + 6 * [pltpu.SemaphoreType.DMA]
        ),
    )
    return fn(ktab, qe, src, dst, t, msg, wt, bt)


def _vagg_body(vlo_hbm, vhi_hbm, ex_hbm, src_hbm, dst_hbm, out_hbm,
               srcv0, dstv0, exv0, srcv1, dstv1, exv1,
               srcv2, dstv2, exv2, srcv3, dstv3, exv3,
               vv0, vv1, outv0, outv1, zb, accum,
               semL0, semL1, semG0, semG1, semS0, semS1):
    c = lax.axis_index("c")
    s = lax.axis_index("s")

    srcv = (srcv0, srcv1, srcv2, srcv3)
    dstv = (dstv0, dstv1, dstv2, dstv3)
    exv = (exv0, exv1, exv2, exv3)
    vv = (vv0, vv1)
    outv = (outv0, outv1)
    semL = (semL0, semL1)
    semG = (semG0, semG1)
    semS = (semS0, semS1)

    zeros16 = jnp.zeros((16,), jnp.float32)

    def zrow(j, carry):
        for kk in range(V_D // 16):
            zb[j, pl.ds(kk * 16, 16)] = zeros16
        return carry

    lax.fori_loop(0, CPR, zrow, 0)
    for j in range(NCP):
        pltpu.sync_copy(zb, accum.at[pl.ds(s * RPT + j * CPR, CPR)])
    plsc.subcore_barrier()

    def lin_issue(ci, l4, l2):
        cc = jnp.minimum(ci, NCH2 - 1)
        row = s * 2 + cc // NCH
        base = (cc % NCH) * C
        return [
            pltpu.async_copy(src_hbm.at[row, pl.ds(base, C)], srcv[l4], semL[l2]),
            pltpu.async_copy(dst_hbm.at[row, pl.ds(base, C)], dstv[l4], semL[l2]),
            pltpu.async_copy(ex_hbm.at[row, pl.ds(base, C)], exv[l4].at[pl.ds(0, C)], semL[l2]),
        ]

    def lin_drain(l4, l2):
        pltpu.make_async_copy(src_hbm.at[0, pl.ds(0, C)], srcv[l4], semL[l2]).wait()
        pltpu.make_async_copy(dst_hbm.at[0, pl.ds(0, C)], dstv[l4], semL[l2]).wait()
        pltpu.make_async_copy(ex_hbm.at[0, pl.ds(0, C)], exv[l4].at[pl.ds(0, C)], semL[l2]).wait()

    def gat_issue(l4, k2):
        @pl.when(c == 0)
        def _():
            pltpu.async_copy(vlo_hbm.at[srcv[l4]], vv[k2], semG[k2])

        @pl.when(c == 1)
        def _():
            pltpu.async_copy(vhi_hbm.at[srcv[l4]], vv[k2], semG[k2])

    def gat_drain(l4, k2):
        pltpu.make_async_copy(vlo_hbm.at[srcv[l4]], vv[k2], semG[k2]).wait()

    def sca_issue(l4, k2):
        pltpu.async_copy(outv[k2], accum.at[dstv[l4]], semS[k2], add=True)

    def sca_drain(l4, k2):
        pltpu.make_async_copy(outv[k2], accum.at[dstv[l4]], semS[k2]).wait()

    _dn = lax.GatherDimensionNumbers(offset_dims=(), collapsed_slice_dims=(0,),
                                     start_index_map=(0,))

    def _bcast(v, j):
        idx = jnp.full((16, 1), j, jnp.int32)
        return lax.gather(v, idx, _dn, slice_sizes=(1,),
                          mode=lax.GatherScatterMode.PROMISE_IN_BOUNDS)

    def compute(l4, k2):
        ev = exv[l4]
        va = vv[k2]
        ov = outv[k2]

        def group(g, gcarry):
            gi = g * 16
            exg = ev[pl.ds(gi, 16)]
            for j in range(16):
                e = gi + j
                exb = _bcast(exg, j)
                for r in range(V_D // 16):
                    ov[e, pl.ds(16 * r, 16)] = exb * va[e, pl.ds(16 * r, 16)]
            return gcarry

        lax.fori_loop(0, G, group, 0)

    lin_issue(0, 0, 0)
    lin_issue(1, 1, 1)
    lin_drain(0, 0)
    gat_issue(0, 0)

    def quad(q4, carry):
        i0 = 4 * q4
        for j in range(4):
            i = i0 + j
            l4 = j
            k2 = j % 2

            @pl.when(i >= 2)
            def _():
                sca_drain((j + 2) % 4, k2)

            la = lin_issue(i + 2, (j + 2) % 4, k2)
            lin_drain((j + 1) % 4, (j + 1) % 2)
            gat_issue((j + 1) % 4, (j + 1) % 2)
            gat_drain(l4, k2)
            compute(l4, k2)
            sca_issue(l4, k2)
            del la
        return carry

    lax.fori_loop(0, NCH2 // 4, quad, 0)
    sca_drain(2, 0)
    sca_drain(3, 1)
    lin_drain(1, 1)
    gat_drain(0, 0)

    plsc.subcore_barrier()
    for j in range(NCP):
        r0 = s * RPT + j * CPR
        pltpu.sync_copy(accum.at[pl.ds(r0, CPR)], zb)
        pltpu.sync_copy(zb, out_hbm.at[c, pl.ds(r0, CPR)])


def _run_vagg(vlo, vhi, ex, src, dst):
    f32 = jnp.float32
    mesh = plsc.VectorSubcoreMesh(core_axis_name="c", subcore_axis_name="s",
                                  num_cores=NC, num_subcores=NS)
    fn = pl.kernel(
        _vagg_body,
        out_type=jax.ShapeDtypeStruct((NC, N2, V_D), f32),
        mesh=mesh,
        compiler_params=pltpu.CompilerParams(use_tc_tiling_on_sc=False),
        scratch_types=(
            4 * [
                pltpu.VMEM((C,), jnp.int32),
                pltpu.VMEM((C,), jnp.int32),
                pltpu.VMEM((C + 16,), f32),
            ]
            + 2 * [
                pltpu.VMEM((C, V_D), f32),
            ]
            + 2 * [
                pltpu.VMEM((C, V_D), f32),
            ]
            + [
                pltpu.VMEM((CPR, V_D), f32),
                pltpu.VMEM_SHARED((N2, V_D), f32),
            ]
            + 6 * [pltpu.SemaphoreType.DMA]
        ),
    )
    return fn(vlo, vhi, ex, src, dst)


def _post_body(sm_ref, vagg_ref, enc_ref, We_ref, be_row_ref, Wa_ref,
               ba_row_ref, out_ref):
    f32 = jnp.float32
    S = sm_ref[0] + sm_ref[1]
    Sm = S[:, 0:16]
    St = S[:, 16:32]
    Sd = S[:, 32:33]
    Sv = jnp.concatenate([vagg_ref[0], vagg_ref[1]], axis=1)
    We = We_ref[...]
    dotT = lambda a, w: lax.dot_general(a, w, (((1,), (1,)), ((), ())),
                                        preferred_element_type=f32)
    num = Sv + dotT(Sm, We[:, 0:16]) + dotT(St, We[:, 16:32]) \
        + Sd * be_row_ref[...]
    conv = num / (Sd + 1e-16)
    enc = enc_ref[...]
    Wa = Wa_ref[...]
    lin = dotT(enc, Wa) - jnp.dot(enc, Wa, preferred_element_type=f32) \
        - GAMMA * enc
    h = jnp.tanh(lin + conv + ba_row_ref[...])
    out_ref[...] = jnp.tanh(enc + EPSILON * h)


def _run_post(sm, vagg, enc, We, be_row, Wa, ba_row):
    f32 = jnp.float32
    return pl.pallas_call(
        _post_body,
        grid=(GRID,),
        in_specs=[pl.BlockSpec((NC, BR, SM_D), lambda i: (0, i, 0)),
                  pl.BlockSpec((NC, BR, V_D), lambda i: (0, i, 0)),
                  pl.BlockSpec((BR, MEM), lambda i: (i, 0)),
                  pl.BlockSpec((MEM, 32), lambda i: (0, 0)),
                  pl.BlockSpec((1, MEM), lambda i: (0, 0)),
                  pl.BlockSpec((MEM, MEM), lambda i: (0, 0)),
                  pl.BlockSpec((1, MEM), lambda i: (0, 0))],
        out_specs=pl.BlockSpec((BR, MEM), lambda i: (i, 0)),
        out_shape=jax.ShapeDtypeStruct((N, MEM), f32),
    )(sm, vagg, enc, We, be_row, Wa, ba_row)


def kernel(x, last_update, edge_index, t, msg, W_time, b_time, W_enc, b_enc,
           Wq, bq, Wk, bk, Wv, bv, We, be, W_asym, b_asym):
    PN = N2 - N
    PE = E2 - E
    x2 = jnp.concatenate([x, jnp.zeros((PN, MEM), jnp.float32)], axis=0)
    lu2 = jnp.concatenate([last_update, jnp.zeros((PN,), jnp.float32)]
                          ).reshape(N2, 1)
    row = lambda b: b.reshape(1, MEM)
    ktab, vlo, vhi, qe, enc = _run_pre(x2, lu2, W_enc, row(b_enc), Wq, row(bq),
                                       Wk, row(bk), Wv, row(bv), We,
                                       be.reshape(MEM, 1))
    src = jnp.concatenate([edge_index[0], jnp.zeros((PE,), jnp.int32)]
                          ).reshape(NW, EPW)
    dst = jnp.concatenate([edge_index[1], jnp.full((PE,), N2 - 1, jnp.int32)]
                          ).reshape(NW, EPW)
    t2 = jnp.concatenate([t, jnp.zeros((PE,), jnp.float32)]).reshape(NW, EPW)
    msg2 = jnp.concatenate([msg, jnp.zeros((PE, 16), jnp.float32)], axis=0)
    ex, sm = _run_alpha(ktab, qe, src, dst, t2, msg2,
                        W_time.reshape(1, 16), b_time.reshape(1, 16))
    vagg = _run_vagg(vlo, vhi, ex, src, dst)
    return _run_post(sm, vagg, enc, We, row(be), W_asym, row(b_asym))


# speedup vs baseline: 7.5132x; 1.0489x over previous
import jax
import jax.numpy as jnp
from jax import lax
from jax.experimental import pallas as pl
from jax.experimental.pallas import tpu as pltpu
from jax.experimental.pallas import tpu_sc as plsc

N = 10000
E = 320000
MEM = 128
GAMMA = 0.1
EPSILON = 1.0
INV_SQRT_MEM = 1.0 / (128.0 ** 0.5)

K_D = 144
QE_D = 176
SM_D = 48
V_D = 64

NC = 2
NS = 16
NW = NC * NS
N2 = 10240
E2 = 327680
EPW = E2 // NW
C = 80
NCH = EPW // C
G = C // 16
RPT = N2 // NS
CPR = 128
NCP = RPT // CPR
NCH2 = 2 * NCH

BRP = 256
GRIDP = N2 // BRP
BR = 200
GRID = N // BR

TWO_PI = 6.283185307179586
INV_2PI = 1.0 / TWO_PI
COS_COEF = (0.9999999999193508, -19.739208758208584, 64.93939011340913,
            -85.45668538180254, 60.24246470872289, -26.406761080377983,
            7.806608463960106, -1.4609479689305238)


def _cos_poly(z):
    u = z * INV_2PI
    n = u.astype(jnp.int32).astype(jnp.float32)
    f = u - n
    f = jnp.where(f > 0.5, f - 1.0, f)
    f = jnp.where(f < -0.5, f + 1.0, f)
    y = f * f
    c = COS_COEF
    y2 = y * y
    y4 = y2 * y2
    p01 = c[0] + c[1] * y
    p23 = c[2] + c[3] * y
    p45 = c[4] + c[5] * y
    p67 = c[6] + c[7] * y
    return (p01 + p23 * y2) + (p45 + p67 * y2) * y4


def _pre_body(x_ref, lu_ref, W_enc_ref, b_enc_ref, Wq_ref, bq_ref, Wk_ref,
              bk_ref, Wv_ref, bv_ref, We_ref, be_col_ref,
              k_ref, vlo_ref, vhi_ref, qe_ref, enc_ref):
    f32 = jnp.float32
    x = x_ref[...]
    dotT = lambda a, w: lax.dot_general(a, w, (((1,), (1,)), ((), ())),
                                        preferred_element_type=f32)
    enc = dotT(x, W_enc_ref[...]) + b_enc_ref[...]
    q = dotT(enc, Wq_ref[...]) + bq_ref[...]
    k = dotT(enc, Wk_ref[...]) + bk_ref[...]
    v = dotT(enc, Wv_ref[...]) + bv_ref[...]
    em = jnp.dot(q, We_ref[...], preferred_element_type=f32)
    qb = jnp.dot(q, be_col_ref[...], preferred_element_type=f32)
    pad = jnp.zeros((BRP, 15), dtype=f32)
    k_ref[...] = jnp.concatenate([k, lu_ref[...], pad], axis=1)
    vlo_ref[...] = v[:, 0:V_D]
    vhi_ref[...] = v[:, V_D:MEM]
    qe_ref[...] = jnp.concatenate([q, em, qb, pad], axis=1)
    enc_ref[...] = enc


def _run_pre(x, lu2, W_enc, b_enc, Wq, bq, Wk, bk, Wv, bv, We, be_col):
    f32 = jnp.float32
    row = lambda d: pl.BlockSpec((BRP, d), lambda i: (i, 0))
    full = lambda a, b: pl.BlockSpec((a, b), lambda i: (0, 0))
    return pl.pallas_call(
        _pre_body,
        grid=(GRIDP,),
        in_specs=[row(MEM), row(1), full(MEM, MEM), full(1, MEM),
                  full(MEM, MEM), full(1, MEM), full(MEM, MEM), full(1, MEM),
                  full(MEM, MEM), full(1, MEM), full(MEM, 32), full(MEM, 1)],
        out_specs=[row(K_D), row(V_D), row(V_D), row(QE_D), row(MEM)],
        out_shape=[jax.ShapeDtypeStruct((N2, K_D), f32),
                   jax.ShapeDtypeStruct((N2, V_D), f32),
                   jax.ShapeDtypeStruct((N2, V_D), f32),
                   jax.ShapeDtypeStruct((N2, QE_D), f32),
                   jax.ShapeDtypeStruct((N2, MEM), f32)],
    )(x, lu2, W_enc, b_enc, Wq, bq, Wk, bk, Wv, bv, We, be_col)


def _alpha_body(k_hbm, qe_hbm, src_hbm, dst_hbm, t_hbm, msg_hbm, wt_hbm,
                bt_hbm, ex_hbm, out_hbm,
                srcv0, dstv0, tv0, msgv0, srcv1, dstv1, tv1, msgv1,
                srcv2, dstv2, tv2, msgv2, srcv3, dstv3, tv3, msgv3,
                kvv0, qev0, kvv1, qev1, outv0, outv1,
                exall, wtv, btv, zb, accum,
                semL0, semL1, semG0, semG1, semS0, semS1):
    c = lax.axis_index("c")
    s = lax.axis_index("s")
    wid = s * NC + c

    srcv = (srcv0, srcv1, srcv2, srcv3)
    dstv = (dstv0, dstv1, dstv2, dstv3)
    tv = (tv0, tv1, tv2, tv3)
    msgv = (msgv0, msgv1, msgv2, msgv3)
    kvv = (kvv0, kvv1)
    qev = (qev0, qev1)
    outv = (outv0, outv1)
    semL = (semL0, semL1)
    semG = (semG0, semG1)
    semS = (semS0, semS1)

    pltpu.sync_copy(wt_hbm, wtv)
    pltpu.sync_copy(bt_hbm, btv)
    wt = wtv[0, pl.ds(0, 16)]
    bt = btv[0, pl.ds(0, 16)]
    lane = lax.iota(jnp.int32, 16)
    unit = jnp.where(lane == 0, 1.0, 0.0).astype(jnp.float32)
    _dn = lax.GatherDimensionNumbers(offset_dims=(), collapsed_slice_dims=(0,),
                                     start_index_map=(0,))
    _perms = [(lane ^ m)[:, None] for m in (8, 4, 2, 1)]

    def _allsum(a):
        for p in _perms:
            a = a + lax.gather(a, p, _dn, slice_sizes=(1,),
                               mode=lax.GatherScatterMode.PROMISE_IN_BOUNDS)
        return a

    def _bcast(v, j):
        idx = jnp.full((16, 1), j, jnp.int32)
        return lax.gather(v, idx, _dn, slice_sizes=(1,),
                          mode=lax.GatherScatterMode.PROMISE_IN_BOUNDS)

    zeros16 = jnp.zeros((16,), jnp.float32)

    def zrow(j, carry):
        for kk in range(SM_D // 16):
            zb[j, pl.ds(kk * 16, 16)] = zeros16
        return carry

    lax.fori_loop(0, CPR, zrow, 0)
    for j in range(NCP):
        pltpu.sync_copy(zb, accum.at[pl.ds(s * RPT + j * CPR, CPR)])
    plsc.subcore_barrier()

    def lin_issue(ci, l4, l2):
        base = jnp.minimum(ci, NCH - 1) * C
        return [
            pltpu.async_copy(src_hbm.at[wid, pl.ds(base, C)], srcv[l4], semL[l2]),
            pltpu.async_copy(dst_hbm.at[wid, pl.ds(base, C)], dstv[l4], semL[l2]),
            pltpu.async_copy(t_hbm.at[wid, pl.ds(base, C)], tv[l4].at[pl.ds(0, C)], semL[l2]),
            pltpu.async_copy(msg_hbm.at[pl.ds(wid * EPW + base, C)], msgv[l4], semL[l2]),
        ]

    def lin_drain(l4, l2):
        pltpu.make_async_copy(src_hbm.at[wid, pl.ds(0, C)], srcv[l4], semL[l2]).wait()
        pltpu.make_async_copy(dst_hbm.at[wid, pl.ds(0, C)], dstv[l4], semL[l2]).wait()
        pltpu.make_async_copy(t_hbm.at[wid, pl.ds(0, C)], tv[l4].at[pl.ds(0, C)], semL[l2]).wait()
        pltpu.make_async_copy(msg_hbm.at[pl.ds(0, C)], msgv[l4], semL[l2]).wait()

    def gat_issue(l4, k2):
        pltpu.async_copy(k_hbm.at[srcv[l4]], kvv[k2], semG[k2])
        pltpu.async_copy(qe_hbm.at[dstv[l4]], qev[k2], semG[k2])

    def gat_drain(l4, k2):
        pltpu.make_async_copy(k_hbm.at[srcv[l4]], kvv[k2], semG[k2]).wait()
        pltpu.make_async_copy(qe_hbm.at[dstv[l4]], qev[k2], semG[k2]).wait()

    def sca_issue(l4, k2):
        pltpu.async_copy(outv[k2], accum.at[dstv[l4]], semS[k2], add=True)

    def sca_drain(l4, k2):
        pltpu.make_async_copy(outv[k2], accum.at[dstv[l4]], semS[k2]).wait()

    def compute(ci, l4, k2):
        kv = kvv[k2]
        qe = qev[k2]
        ov = outv[k2]
        mv = msgv[l4]
        tt = tv[l4]

        def group(g):
            gi = g * 16
            t_g = tt[pl.ds(gi, 16)]
            exg = zeros16
            for j in range(16):
                e = gi + j
                acc = qe[e, pl.ds(0, 16)] * kv[e, pl.ds(0, 16)]
                for r in range(1, 8):
                    acc = acc + qe[e, pl.ds(16 * r, 16)] * kv[e, pl.ds(16 * r, 16)]
                msg_v = mv[e, pl.ds(0, 16)]
                rel = jnp.abs(_bcast(kv[e, pl.ds(128, 16)], 0) - _bcast(t_g, j))
                te = _cos_poly(rel * wt + bt)
                acc = acc + qe[e, pl.ds(128, 16)] * msg_v
                acc = acc + qe[e, pl.ds(144, 16)] * te
                acc = acc + qe[e, pl.ds(160, 16)]
                ex = jnp.exp(_allsum(acc) * INV_SQRT_MEM)
                ov[e, pl.ds(0, 16)] = ex * msg_v
                ov[e, pl.ds(16, 16)] = ex * te
                ov[e, pl.ds(32, 16)] = ex * unit
                exg = jnp.where(lane == j, ex, exg)
            exall[pl.ds(ci * C + gi, 16)] = exg

        plsc.parallel_loop(0, G, 1)(group)

    lin_issue(0, 0, 0)
    lin_issue(1, 1, 1)
    lin_drain(0, 0)
    gat_issue(0, 0)

    def quad(q4, carry):
        i0 = 4 * q4
        for j in range(4):
            i = i0 + j
            l4 = j
            k2 = j % 2

            @pl.when(i >= 2)
            def _():
                sca_drain((j + 2) % 4, k2)

            la = lin_issue(i + 2, (j + 2) % 4, k2)
            lin_drain((j + 1) % 4, (j + 1) % 2)
            gat_issue((j + 1) % 4, (j + 1) % 2)
            gat_drain(l4, k2)
            compute(i, l4, k2)
            sca_issue(l4, k2)
            del la
        return carry

    lax.fori_loop(0, NCH // 4, quad, 0)
    sca_drain(2, 0)
    sca_drain(3, 1)
    lin_drain(1, 1)
    gat_drain(0, 0)

    pltpu.sync_copy(exall, ex_hbm.at[wid])

    plsc.subcore_barrier()
    for j in range(NCP):
        r0 = s * RPT + j * CPR
        pltpu.sync_copy(accum.at[pl.ds(r0, CPR)], zb)
        pltpu.sync_copy(zb, out_hbm.at[c, pl.ds(r0, CPR)])


def _run_alpha(ktab, qe, src, dst, t, msg, wt, bt):
    f32 = jnp.float32
    mesh = plsc.VectorSubcoreMesh(core_axis_name="c", subcore_axis_name="s",
                                  num_cores=NC, num_subcores=NS)
    fn = pl.kernel(
        _alpha_body,
        out_type=[jax.ShapeDtypeStruct((NW, EPW), f32),
                  jax.ShapeDtypeStruct((NC, N2, SM_D), f32)],
        mesh=mesh,
        compiler_params=pltpu.CompilerParams(use_tc_tiling_on_sc=False),
        scratch_types=(
            4 * [
                pltpu.VMEM((C,), jnp.int32),
                pltpu.VMEM((C,), jnp.int32),
                pltpu.VMEM((C + 16,), f32),
                pltpu.VMEM((C, 16), f32),
            ]
            + 2 * [
                pltpu.VMEM((C, K_D), f32),
                pltpu.VMEM((C, QE_D), f32),
            ]
            + 2 * [
                pltpu.VMEM((C, SM_D), f32),
            ]
            + [
                pltpu.VMEM((EPW,), f32),
                pltpu.VMEM((1, 16), f32),
                pltpu.VMEM((1, 16), f32),
                pltpu.VMEM((CPR, SM_D), f32),
                pltpu.VMEM_SHARED((N2, SM_D), f32),
            ]
            + 6 * [pltpu.SemaphoreType.DMA]
        ),
    )
    return fn(ktab, qe, src, dst, t, msg, wt, bt)


def _vagg_body(vlo_hbm, vhi_hbm, ex_hbm, src_hbm, dst_hbm, out_hbm,
               srcv0, dstv0, exv0, srcv1, dstv1, exv1,
               srcv2, dstv2, exv2, srcv3, dstv3, exv3,
               vv0, vv1, outv0, outv1, zb, accum,
               semL0, semL1, semG0, semG1, semS0, semS1):
    c = lax.axis_index("c")
    s = lax.axis_index("s")

    srcv = (srcv0, srcv1, srcv2, srcv3)
    dstv = (dstv0, dstv1, dstv2, dstv3)
    exv = (exv0, exv1, exv2, exv3)
    vv = (vv0, vv1)
    outv = (outv0, outv1)
    semL = (semL0, semL1)
    semG = (semG0, semG1)
    semS = (semS0, semS1)

    zeros16 = jnp.zeros((16,), jnp.float32)

    def zrow(j, carry):
        for kk in range(V_D // 16):
            zb[j, pl.ds(kk * 16, 16)] = zeros16
        return carry

    lax.fori_loop(0, CPR, zrow, 0)
    for j in range(NCP):
        pltpu.sync_copy(zb, accum.at[pl.ds(s * RPT + j * CPR, CPR)])
    plsc.subcore_barrier()

    def lin_issue(ci, l4, l2):
        cc = jnp.minimum(ci, NCH2 - 1)
        row = s * 2 + cc // NCH
        base = (cc % NCH) * C
        return [
            pltpu.async_copy(src_hbm.at[row, pl.ds(base, C)], srcv[l4], semL[l2]),
            pltpu.async_copy(dst_hbm.at[row, pl.ds(base, C)], dstv[l4], semL[l2]),
            pltpu.async_copy(ex_hbm.at[row, pl.ds(base, C)], exv[l4].at[pl.ds(0, C)], semL[l2]),
        ]

    def lin_drain(l4, l2):
        pltpu.make_async_copy(src_hbm.at[0, pl.ds(0, C)], srcv[l4], semL[l2]).wait()
        pltpu.make_async_copy(dst_hbm.at[0, pl.ds(0, C)], dstv[l4], semL[l2]).wait()
        pltpu.make_async_copy(ex_hbm.at[0, pl.ds(0, C)], exv[l4].at[pl.ds(0, C)], semL[l2]).wait()

    def gat_issue(l4, k2):
        @pl.when(c == 0)
        def _():
            pltpu.async_copy(vlo_hbm.at[srcv[l4]], vv[k2], semG[k2])

        @pl.when(c == 1)
        def _():
            pltpu.async_copy(vhi_hbm.at[srcv[l4]], vv[k2], semG[k2])

    def gat_drain(l4, k2):
        pltpu.make_async_copy(vlo_hbm.at[srcv[l4]], vv[k2], semG[k2]).wait()

    def sca_issue(l4, k2):
        pltpu.async_copy(outv[k2], accum.at[dstv[l4]], semS[k2], add=True)

    def sca_drain(l4, k2):
        pltpu.make_async_copy(outv[k2], accum.at[dstv[l4]], semS[k2]).wait()

    _dn = lax.GatherDimensionNumbers(offset_dims=(), collapsed_slice_dims=(0,),
                                     start_index_map=(0,))

    def _bcast(v, j):
        idx = jnp.full((16, 1), j, jnp.int32)
        return lax.gather(v, idx, _dn, slice_sizes=(1,),
                          mode=lax.GatherScatterMode.PROMISE_IN_BOUNDS)

    def compute(l4, k2):
        ev = exv[l4]
        va = vv[k2]
        ov = outv[k2]

        def group(g):
            gi = g * 16
            exg = ev[pl.ds(gi, 16)]
            for j in range(16):
                e = gi + j
                exb = _bcast(exg, j)
                for r in range(V_D // 16):
                    ov[e, pl.ds(16 * r, 16)] = exb * va[e, pl.ds(16 * r, 16)]

        plsc.parallel_loop(0, G, 1)(group)

    lin_issue(0, 0, 0)
    lin_issue(1, 1, 1)
    lin_drain(0, 0)
    gat_issue(0, 0)

    def quad(q4, carry):
        i0 = 4 * q4
        for j in range(4):
            i = i0 + j
            l4 = j
            k2 = j % 2

            @pl.when(i >= 2)
            def _():
                sca_drain((j + 2) % 4, k2)

            la = lin_issue(i + 2, (j + 2) % 4, k2)
            lin_drain((j + 1) % 4, (j + 1) % 2)
            gat_issue((j + 1) % 4, (j + 1) % 2)
            gat_drain(l4, k2)
            compute(l4, k2)
            sca_issue(l4, k2)
            del la
        return carry

    lax.fori_loop(0, NCH2 // 4, quad, 0)
    sca_drain(2, 0)
    sca_drain(3, 1)
    lin_drain(1, 1)
    gat_drain(0, 0)

    plsc.subcore_barrier()
    for j in range(NCP):
        r0 = s * RPT + j * CPR
        pltpu.sync_copy(accum.at[pl.ds(r0, CPR)], zb)
        pltpu.sync_copy(zb, out_hbm.at[c, pl.ds(r0, CPR)])


def _run_vagg(vlo, vhi, ex, src, dst):
    f32 = jnp.float32
    mesh = plsc.VectorSubcoreMesh(core_axis_name="c", subcore_axis_name="s",
                                  num_cores=NC, num_subcores=NS)
    fn = pl.kernel(
        _vagg_body,
        out_type=jax.ShapeDtypeStruct((NC, N2, V_D), f32),
        mesh=mesh,
        compiler_params=pltpu.CompilerParams(use_tc_tiling_on_sc=False),
        scratch_types=(
            4 * [
                pltpu.VMEM((C,), jnp.int32),
                pltpu.VMEM((C,), jnp.int32),
                pltpu.VMEM((C + 16,), f32),
            ]
            + 2 * [
                pltpu.VMEM((C, V_D), f32),
            ]
            + 2 * [
                pltpu.VMEM((C, V_D), f32),
            ]
            + [
                pltpu.VMEM((CPR, V_D), f32),
                pltpu.VMEM_SHARED((N2, V_D), f32),
            ]
            + 6 * [pltpu.SemaphoreType.DMA]
        ),
    )
    return fn(vlo, vhi, ex, src, dst)


def _post_body(sm_ref, vagg_ref, enc_ref, We_ref, be_row_ref, Wa_ref,
               ba_row_ref, out_ref):
    f32 = jnp.float32
    S = sm_ref[0] + sm_ref[1]
    Sm = S[:, 0:16]
    St = S[:, 16:32]
    Sd = S[:, 32:33]
    Sv = jnp.concatenate([vagg_ref[0], vagg_ref[1]], axis=1)
    We = We_ref[...]
    dotT = lambda a, w: lax.dot_general(a, w, (((1,), (1,)), ((), ())),
                                        preferred_element_type=f32)
    num = Sv + dotT(Sm, We[:, 0:16]) + dotT(St, We[:, 16:32]) \
        + Sd * be_row_ref[...]
    conv = num / (Sd + 1e-16)
    enc = enc_ref[...]
    Wa = Wa_ref[...]
    lin = dotT(enc, Wa) - jnp.dot(enc, Wa, preferred_element_type=f32) \
        - GAMMA * enc
    h = jnp.tanh(lin + conv + ba_row_ref[...])
    out_ref[...] = jnp.tanh(enc + EPSILON * h)


def _run_post(sm, vagg, enc, We, be_row, Wa, ba_row):
    f32 = jnp.float32
    return pl.pallas_call(
        _post_body,
        grid=(GRID,),
        in_specs=[pl.BlockSpec((NC, BR, SM_D), lambda i: (0, i, 0)),
                  pl.BlockSpec((NC, BR, V_D), lambda i: (0, i, 0)),
                  pl.BlockSpec((BR, MEM), lambda i: (i, 0)),
                  pl.BlockSpec((MEM, 32), lambda i: (0, 0)),
                  pl.BlockSpec((1, MEM), lambda i: (0, 0)),
                  pl.BlockSpec((MEM, MEM), lambda i: (0, 0)),
                  pl.BlockSpec((1, MEM), lambda i: (0, 0))],
        out_specs=pl.BlockSpec((BR, MEM), lambda i: (i, 0)),
        out_shape=jax.ShapeDtypeStruct((N, MEM), f32),
    )(sm, vagg, enc, We, be_row, Wa, ba_row)


def kernel(x, last_update, edge_index, t, msg, W_time, b_time, W_enc, b_enc,
           Wq, bq, Wk, bk, Wv, bv, We, be, W_asym, b_asym):
    PN = N2 - N
    PE = E2 - E
    x2 = jnp.concatenate([x, jnp.zeros((PN, MEM), jnp.float32)], axis=0)
    lu2 = jnp.concatenate([last_update, jnp.zeros((PN,), jnp.float32)]
                          ).reshape(N2, 1)
    row = lambda b: b.reshape(1, MEM)
    ktab, vlo, vhi, qe, enc = _run_pre(x2, lu2, W_enc, row(b_enc), Wq, row(bq),
                                       Wk, row(bk), Wv, row(bv), We,
                                       be.reshape(MEM, 1))
    src = jnp.concatenate([edge_index[0], jnp.zeros((PE,), jnp.int32)]
                          ).reshape(NW, EPW)
    dst = jnp.concatenate([edge_index[1], jnp.full((PE,), N2 - 1, jnp.int32)]
                          ).reshape(NW, EPW)
    t2 = jnp.concatenate([t, jnp.zeros((PE,), jnp.float32)]).reshape(NW, EPW)
    msg2 = jnp.concatenate([msg, jnp.zeros((PE, 16), jnp.float32)], axis=0)
    ex, sm = _run_alpha(ktab, qe, src, dst, t2, msg2,
                        W_time.reshape(1, 16), b_time.reshape(1, 16))
    vagg = _run_vagg(vlo, vhi, ex, src, dst)
    return _run_post(sm, vagg, enc, We, row(be), W_asym, row(b_asym))


# speedup vs baseline: 7.5432x; 1.0040x over previous
import jax
import jax.numpy as jnp
from jax import lax
from jax.experimental import pallas as pl
from jax.experimental.pallas import tpu as pltpu
from jax.experimental.pallas import tpu_sc as plsc

N = 10000
E = 320000
MEM = 128
GAMMA = 0.1
EPSILON = 1.0
INV_SQRT_MEM = 1.0 / (128.0 ** 0.5)

K_D = 144
QE_D = 176
SM_D = 48
V_D = 64

NC = 2
NS = 16
NW = NC * NS
N2 = 10240
E2 = 327680
EPW = E2 // NW
C = 80
NCH = EPW // C
G = C // 16
RPT = N2 // NS
CPR = 128
NCP = RPT // CPR
NCH2 = 2 * NCH

BRP = 256
GRIDP = N2 // BRP
BR = 200
GRID = N // BR

TWO_PI = 6.283185307179586
INV_2PI = 1.0 / TWO_PI
COS_COEF = (0.9999999999193508, -19.739208758208584, 64.93939011340913,
            -85.45668538180254, 60.24246470872289, -26.406761080377983,
            7.806608463960106, -1.4609479689305238)


def _cos_poly(z):
    u = z * INV_2PI
    n = u.astype(jnp.int32).astype(jnp.float32)
    f = u - n
    f = jnp.where(f > 0.5, f - 1.0, f)
    f = jnp.where(f < -0.5, f + 1.0, f)
    y = f * f
    c = COS_COEF
    y2 = y * y
    y4 = y2 * y2
    p01 = c[0] + c[1] * y
    p23 = c[2] + c[3] * y
    p45 = c[4] + c[5] * y
    p67 = c[6] + c[7] * y
    return (p01 + p23 * y2) + (p45 + p67 * y2) * y4


def _pre_body(x_ref, lu_ref, W_enc_ref, b_enc_ref, Wq_ref, bq_ref, Wk_ref,
              bk_ref, Wv_ref, bv_ref, We_ref, be_col_ref,
              k_ref, vlo_ref, vhi_ref, qe_ref, enc_ref):
    f32 = jnp.float32
    x = x_ref[...]
    dotT = lambda a, w: lax.dot_general(a, w, (((1,), (1,)), ((), ())),
                                        preferred_element_type=f32)
    enc = dotT(x, W_enc_ref[...]) + b_enc_ref[...]
    q = dotT(enc, Wq_ref[...]) + bq_ref[...]
    k = dotT(enc, Wk_ref[...]) + bk_ref[...]
    v = dotT(enc, Wv_ref[...]) + bv_ref[...]
    em = jnp.dot(q, We_ref[...], preferred_element_type=f32)
    qb = jnp.dot(q, be_col_ref[...], preferred_element_type=f32)
    pad = jnp.zeros((BRP, 15), dtype=f32)
    k_ref[...] = jnp.concatenate([k, lu_ref[...], pad], axis=1)
    vlo_ref[...] = v[:, 0:V_D]
    vhi_ref[...] = v[:, V_D:MEM]
    qe_ref[...] = jnp.concatenate([q, em, qb, pad], axis=1)
    enc_ref[...] = enc


def _run_pre(x, lu2, W_enc, b_enc, Wq, bq, Wk, bk, Wv, bv, We, be_col):
    f32 = jnp.float32
    row = lambda d: pl.BlockSpec((BRP, d), lambda i: (i, 0))
    full = lambda a, b: pl.BlockSpec((a, b), lambda i: (0, 0))
    return pl.pallas_call(
        _pre_body,
        grid=(GRIDP,),
        in_specs=[row(MEM), row(1), full(MEM, MEM), full(1, MEM),
                  full(MEM, MEM), full(1, MEM), full(MEM, MEM), full(1, MEM),
                  full(MEM, MEM), full(1, MEM), full(MEM, 32), full(MEM, 1)],
        out_specs=[row(K_D), row(V_D), row(V_D), row(QE_D), row(MEM)],
        out_shape=[jax.ShapeDtypeStruct((N2, K_D), f32),
                   jax.ShapeDtypeStruct((N2, V_D), f32),
                   jax.ShapeDtypeStruct((N2, V_D), f32),
                   jax.ShapeDtypeStruct((N2, QE_D), f32),
                   jax.ShapeDtypeStruct((N2, MEM), f32)],
    )(x, lu2, W_enc, b_enc, Wq, bq, Wk, bk, Wv, bv, We, be_col)


def _alpha_body(k_hbm, qe_hbm, src_hbm, dst_hbm, t_hbm, msg_hbm, wt_hbm,
                bt_hbm, ex_hbm, out_hbm,
                srcv0, dstv0, tv0, msgv0, srcv1, dstv1, tv1, msgv1,
                srcv2, dstv2, tv2, msgv2, srcv3, dstv3, tv3, msgv3,
                kvv0, qev0, kvv1, qev1, outv0, outv1,
                exall, wtv, btv, zb, accum,
                semL0, semL1, semG0, semG1, semS0, semS1):
    c = lax.axis_index("c")
    s = lax.axis_index("s")
    wid = s * NC + c

    srcv = (srcv0, srcv1, srcv2, srcv3)
    dstv = (dstv0, dstv1, dstv2, dstv3)
    tv = (tv0, tv1, tv2, tv3)
    msgv = (msgv0, msgv1, msgv2, msgv3)
    kvv = (kvv0, kvv1)
    qev = (qev0, qev1)
    outv = (outv0, outv1)
    semL = (semL0, semL1)
    semG = (semG0, semG1)
    semS = (semS0, semS1)

    pltpu.sync_copy(wt_hbm, wtv)
    pltpu.sync_copy(bt_hbm, btv)
    wt = wtv[0, pl.ds(0, 16)]
    bt = btv[0, pl.ds(0, 16)]
    lane = lax.iota(jnp.int32, 16)
    unit = jnp.where(lane == 0, 1.0, 0.0).astype(jnp.float32)
    _dn = lax.GatherDimensionNumbers(offset_dims=(), collapsed_slice_dims=(0,),
                                     start_index_map=(0,))
    _perms = [(lane ^ m)[:, None] for m in (8, 4, 2, 1)]

    def _allsum(a):
        for p in _perms:
            a = a + lax.gather(a, p, _dn, slice_sizes=(1,),
                               mode=lax.GatherScatterMode.PROMISE_IN_BOUNDS)
        return a

    def _bcast(v, j):
        idx = jnp.full((16, 1), j, jnp.int32)
        return lax.gather(v, idx, _dn, slice_sizes=(1,),
                          mode=lax.GatherScatterMode.PROMISE_IN_BOUNDS)

    zeros16 = jnp.zeros((16,), jnp.float32)

    def zrow(j, carry):
        for kk in range(SM_D // 16):
            zb[j, pl.ds(kk * 16, 16)] = zeros16
        return carry

    lax.fori_loop(0, CPR, zrow, 0)
    for j in range(NCP):
        pltpu.sync_copy(zb, accum.at[pl.ds(s * RPT + j * CPR, CPR)])
    plsc.subcore_barrier()

    def lin_issue(ci, l4, l2):
        base = jnp.minimum(ci, NCH - 1) * C
        return [
            pltpu.async_copy(src_hbm.at[wid, pl.ds(base, C)], srcv[l4], semL[l2]),
            pltpu.async_copy(dst_hbm.at[wid, pl.ds(base, C)], dstv[l4], semL[l2]),
            pltpu.async_copy(t_hbm.at[wid, pl.ds(base, C)], tv[l4].at[pl.ds(0, C)], semL[l2]),
            pltpu.async_copy(msg_hbm.at[pl.ds(wid * EPW + base, C)], msgv[l4], semL[l2]),
        ]

    def lin_drain(l4, l2):
        pltpu.make_async_copy(src_hbm.at[wid, pl.ds(0, C)], srcv[l4], semL[l2]).wait()
        pltpu.make_async_copy(dst_hbm.at[wid, pl.ds(0, C)], dstv[l4], semL[l2]).wait()
        pltpu.make_async_copy(t_hbm.at[wid, pl.ds(0, C)], tv[l4].at[pl.ds(0, C)], semL[l2]).wait()
        pltpu.make_async_copy(msg_hbm.at[pl.ds(0, C)], msgv[l4], semL[l2]).wait()

    def gat_issue(l4, k2):
        pltpu.async_copy(k_hbm.at[srcv[l4]], kvv[k2], semG[k2])
        pltpu.async_copy(qe_hbm.at[dstv[l4]], qev[k2], semG[k2])

    def gat_drain(l4, k2):
        pltpu.make_async_copy(k_hbm.at[srcv[l4]], kvv[k2], semG[k2]).wait()
        pltpu.make_async_copy(qe_hbm.at[dstv[l4]], qev[k2], semG[k2]).wait()

    def sca_issue(l4, k2):
        pltpu.async_copy(outv[k2], accum.at[dstv[l4]], semS[k2], add=True)

    def sca_drain(l4, k2):
        pltpu.make_async_copy(outv[k2], accum.at[dstv[l4]], semS[k2]).wait()

    lane0 = lane == 0

    def compute(ci, l4, k2):
        kv = kvv[k2]
        qe = qev[k2]
        ov = outv[k2]
        mv = msgv[l4]
        tt = tv[l4]

        def edge(e, exg):
            acc = qe[e, pl.ds(0, 16)] * kv[e, pl.ds(0, 16)]
            for r in range(1, 8):
                acc = acc + qe[e, pl.ds(16 * r, 16)] * kv[e, pl.ds(16 * r, 16)]
            msg_v = mv[e, pl.ds(0, 16)]
            rel = jnp.abs(_bcast(kv[e, pl.ds(128, 16)], 0)
                          - _bcast(tt[pl.ds(e, 16)], 0))
            te = _cos_poly(rel * wt + bt)
            acc = acc + qe[e, pl.ds(128, 16)] * msg_v
            acc = acc + qe[e, pl.ds(144, 16)] * te
            acc = acc + qe[e, pl.ds(160, 16)]
            ex = jnp.exp(_allsum(acc) * INV_SQRT_MEM)
            ov[e, pl.ds(0, 16)] = ex * msg_v
            ov[e, pl.ds(16, 16)] = ex * te
            ov[e, pl.ds(32, 16)] = ex * unit
            j = e & 15
            exg = jnp.where(lane == j, ex, exg)

            @pl.when(j == 15)
            def _():
                exall[pl.ds(ci * C + e - 15, 16)] = exg

            return exg

        plsc.parallel_loop(0, C, 1, unroll=16, carry=zeros16)(edge)

    lin_issue(0, 0, 0)
    lin_issue(1, 1, 1)
    lin_drain(0, 0)
    gat_issue(0, 0)

    def quad(q4, carry):
        i0 = 4 * q4
        for j in range(4):
            i = i0 + j
            l4 = j
            k2 = j % 2

            @pl.when(i >= 2)
            def _():
                sca_drain((j + 2) % 4, k2)

            la = lin_issue(i + 2, (j + 2) % 4, k2)
            lin_drain((j + 1) % 4, (j + 1) % 2)
            gat_issue((j + 1) % 4, (j + 1) % 2)
            gat_drain(l4, k2)
            compute(i, l4, k2)
            sca_issue(l4, k2)
            del la
        return carry

    lax.fori_loop(0, NCH // 4, quad, 0)
    sca_drain(2, 0)
    sca_drain(3, 1)
    lin_drain(1, 1)
    gat_drain(0, 0)

    pltpu.sync_copy(exall, ex_hbm.at[wid])

    plsc.subcore_barrier()
    for j in range(NCP):
        r0 = s * RPT + j * CPR
        pltpu.sync_copy(accum.at[pl.ds(r0, CPR)], zb)
        pltpu.sync_copy(zb, out_hbm.at[c, pl.ds(r0, CPR)])


def _run_alpha(ktab, qe, src, dst, t, msg, wt, bt):
    f32 = jnp.float32
    mesh = plsc.VectorSubcoreMesh(core_axis_name="c", subcore_axis_name="s",
                                  num_cores=NC, num_subcores=NS)
    fn = pl.kernel(
        _alpha_body,
        out_type=[jax.ShapeDtypeStruct((NW, EPW), f32),
                  jax.ShapeDtypeStruct((NC, N2, SM_D), f32)],
        mesh=mesh,
        compiler_params=pltpu.CompilerParams(use_tc_tiling_on_sc=False),
        scratch_types=(
            4 * [
                pltpu.VMEM((C,), jnp.int32),
                pltpu.VMEM((C,), jnp.int32),
                pltpu.VMEM((C + 16,), f32),
                pltpu.VMEM((C, 16), f32),
            ]
            + 2 * [
                pltpu.VMEM((C, K_D), f32),
                pltpu.VMEM((C, QE_D), f32),
            ]
            + 2 * [
                pltpu.VMEM((C, SM_D), f32),
            ]
            + [
                pltpu.VMEM((EPW,), f32),
                pltpu.VMEM((1, 16), f32),
                pltpu.VMEM((1, 16), f32),
                pltpu.VMEM((CPR, SM_D), f32),
                pltpu.VMEM_SHARED((N2, SM_D), f32),
            ]
            + 6 * [pltpu.SemaphoreType.DMA]
        ),
    )
    return fn(ktab, qe, src, dst, t, msg, wt, bt)


def _vagg_body(vlo_hbm, vhi_hbm, ex_hbm, src_hbm, dst_hbm, out_hbm,
               srcv0, dstv0, exv0, srcv1, dstv1, exv1,
               srcv2, dstv2, exv2, srcv3, dstv3, exv3,
               vv0, vv1, outv0, outv1, zb, accum,
               semL0, semL1, semG0, semG1, semS0, semS1):
    c = lax.axis_index("c")
    s = lax.axis_index("s")

    srcv = (srcv0, srcv1, srcv2, srcv3)
    dstv = (dstv0, dstv1, dstv2, dstv3)
    exv = (exv0, exv1, exv2, exv3)
    vv = (vv0, vv1)
    outv = (outv0, outv1)
    semL = (semL0, semL1)
    semG = (semG0, semG1)
    semS = (semS0, semS1)

    zeros16 = jnp.zeros((16,), jnp.float32)

    def zrow(j, carry):
        for kk in range(V_D // 16):
            zb[j, pl.ds(kk * 16, 16)] = zeros16
        return carry

    lax.fori_loop(0, CPR, zrow, 0)
    for j in range(NCP):
        pltpu.sync_copy(zb, accum.at[pl.ds(s * RPT + j * CPR, CPR)])
    plsc.subcore_barrier()

    def lin_issue(ci, l4, l2):
        cc = jnp.minimum(ci, NCH2 - 1)
        row = s * 2 + cc // NCH
        base = (cc % NCH) * C
        return [
            pltpu.async_copy(src_hbm.at[row, pl.ds(base, C)], srcv[l4], semL[l2]),
            pltpu.async_copy(dst_hbm.at[row, pl.ds(base, C)], dstv[l4], semL[l2]),
            pltpu.async_copy(ex_hbm.at[row, pl.ds(base, C)], exv[l4].at[pl.ds(0, C)], semL[l2]),
        ]

    def lin_drain(l4, l2):
        pltpu.make_async_copy(src_hbm.at[0, pl.ds(0, C)], srcv[l4], semL[l2]).wait()
        pltpu.make_async_copy(dst_hbm.at[0, pl.ds(0, C)], dstv[l4], semL[l2]).wait()
        pltpu.make_async_copy(ex_hbm.at[0, pl.ds(0, C)], exv[l4].at[pl.ds(0, C)], semL[l2]).wait()

    def gat_issue(l4, k2):
        @pl.when(c == 0)
        def _():
            pltpu.async_copy(vlo_hbm.at[srcv[l4]], vv[k2], semG[k2])

        @pl.when(c == 1)
        def _():
            pltpu.async_copy(vhi_hbm.at[srcv[l4]], vv[k2], semG[k2])

    def gat_drain(l4, k2):
        pltpu.make_async_copy(vlo_hbm.at[srcv[l4]], vv[k2], semG[k2]).wait()

    def sca_issue(l4, k2):
        pltpu.async_copy(outv[k2], accum.at[dstv[l4]], semS[k2], add=True)

    def sca_drain(l4, k2):
        pltpu.make_async_copy(outv[k2], accum.at[dstv[l4]], semS[k2]).wait()

    _dn = lax.GatherDimensionNumbers(offset_dims=(), collapsed_slice_dims=(0,),
                                     start_index_map=(0,))

    def _bcast(v, j):
        idx = jnp.full((16, 1), j, jnp.int32)
        return lax.gather(v, idx, _dn, slice_sizes=(1,),
                          mode=lax.GatherScatterMode.PROMISE_IN_BOUNDS)

    def compute(l4, k2):
        ev = exv[l4]
        va = vv[k2]
        ov = outv[k2]

        def edge(e):
            exb = _bcast(ev[pl.ds(e, 16)], 0)
            for r in range(V_D // 16):
                ov[e, pl.ds(16 * r, 16)] = exb * va[e, pl.ds(16 * r, 16)]

        plsc.parallel_loop(0, C, 1, unroll=16)(edge)

    lin_issue(0, 0, 0)
    lin_issue(1, 1, 1)
    lin_drain(0, 0)
    gat_issue(0, 0)

    def quad(q4, carry):
        i0 = 4 * q4
        for j in range(4):
            i = i0 + j
            l4 = j
            k2 = j % 2

            @pl.when(i >= 2)
            def _():
                sca_drain((j + 2) % 4, k2)

            la = lin_issue(i + 2, (j + 2) % 4, k2)
            lin_drain((j + 1) % 4, (j + 1) % 2)
            gat_issue((j + 1) % 4, (j + 1) % 2)
            gat_drain(l4, k2)
            compute(l4, k2)
            sca_issue(l4, k2)
            del la
        return carry

    lax.fori_loop(0, NCH2 // 4, quad, 0)
    sca_drain(2, 0)
    sca_drain(3, 1)
    lin_drain(1, 1)
    gat_drain(0, 0)

    plsc.subcore_barrier()
    for j in range(NCP):
        r0 = s * RPT + j * CPR
        pltpu.sync_copy(accum.at[pl.ds(r0, CPR)], zb)
        pltpu.sync_copy(zb, out_hbm.at[c, pl.ds(r0, CPR)])


def _run_vagg(vlo, vhi, ex, src, dst):
    f32 = jnp.float32
    mesh = plsc.VectorSubcoreMesh(core_axis_name="c", subcore_axis_name="s",
                                  num_cores=NC, num_subcores=NS)
    fn = pl.kernel(
        _vagg_body,
        out_type=jax.ShapeDtypeStruct((NC, N2, V_D), f32),
        mesh=mesh,
        compiler_params=pltpu.CompilerParams(use_tc_tiling_on_sc=False),
        scratch_types=(
            4 * [
                pltpu.VMEM((C,), jnp.int32),
                pltpu.VMEM((C,), jnp.int32),
                pltpu.VMEM((C + 16,), f32),
            ]
            + 2 * [
                pltpu.VMEM((C, V_D), f32),
            ]
            + 2 * [
                pltpu.VMEM((C, V_D), f32),
            ]
            + [
                pltpu.VMEM((CPR, V_D), f32),
                pltpu.VMEM_SHARED((N2, V_D), f32),
            ]
            + 6 * [pltpu.SemaphoreType.DMA]
        ),
    )
    return fn(vlo, vhi, ex, src, dst)


def _post_body(sm_ref, vagg_ref, enc_ref, We_ref, be_row_ref, Wa_ref,
               ba_row_ref, out_ref):
    f32 = jnp.float32
    S = sm_ref[0] + sm_ref[1]
    Sm = S[:, 0:16]
    St = S[:, 16:32]
    Sd = S[:, 32:33]
    Sv = jnp.concatenate([vagg_ref[0], vagg_ref[1]], axis=1)
    We = We_ref[...]
    dotT = lambda a, w: lax.dot_general(a, w, (((1,), (1,)), ((), ())),
                                        preferred_element_type=f32)
    num = Sv + dotT(Sm, We[:, 0:16]) + dotT(St, We[:, 16:32]) \
        + Sd * be_row_ref[...]
    conv = num / (Sd + 1e-16)
    enc = enc_ref[...]
    Wa = Wa_ref[...]
    lin = dotT(enc, Wa) - jnp.dot(enc, Wa, preferred_element_type=f32) \
        - GAMMA * enc
    h = jnp.tanh(lin + conv + ba_row_ref[...])
    out_ref[...] = jnp.tanh(enc + EPSILON * h)


def _run_post(sm, vagg, enc, We, be_row, Wa, ba_row):
    f32 = jnp.float32
    return pl.pallas_call(
        _post_body,
        grid=(GRID,),
        in_specs=[pl.BlockSpec((NC, BR, SM_D), lambda i: (0, i, 0)),
                  pl.BlockSpec((NC, BR, V_D), lambda i: (0, i, 0)),
                  pl.BlockSpec((BR, MEM), lambda i: (i, 0)),
                  pl.BlockSpec((MEM, 32), lambda i: (0, 0)),
                  pl.BlockSpec((1, MEM), lambda i: (0, 0)),
                  pl.BlockSpec((MEM, MEM), lambda i: (0, 0)),
                  pl.BlockSpec((1, MEM), lambda i: (0, 0))],
        out_specs=pl.BlockSpec((BR, MEM), lambda i: (i, 0)),
        out_shape=jax.ShapeDtypeStruct((N, MEM), f32),
    )(sm, vagg, enc, We, be_row, Wa, ba_row)


def kernel(x, last_update, edge_index, t, msg, W_time, b_time, W_enc, b_enc,
           Wq, bq, Wk, bk, Wv, bv, We, be, W_asym, b_asym):
    PN = N2 - N
    PE = E2 - E
    x2 = jnp.concatenate([x, jnp.zeros((PN, MEM), jnp.float32)], axis=0)
    lu2 = jnp.concatenate([last_update, jnp.zeros((PN,), jnp.float32)]
                          ).reshape(N2, 1)
    row = lambda b: b.reshape(1, MEM)
    ktab, vlo, vhi, qe, enc = _run_pre(x2, lu2, W_enc, row(b_enc), Wq, row(bq),
                                       Wk, row(bk), Wv, row(bv), We,
                                       be.reshape(MEM, 1))
    src = jnp.concatenate([edge_index[0], jnp.zeros((PE,), jnp.int32)]
                          ).reshape(NW, EPW)
    dst = jnp.concatenate([edge_index[1], jnp.full((PE,), N2 - 1, jnp.int32)]
                          ).reshape(NW, EPW)
    t2 = jnp.concatenate([t, jnp.zeros((PE,), jnp.float32)]).reshape(NW, EPW)
    msg2 = jnp.concatenate([msg, jnp.zeros((PE, 16), jnp.float32)], axis=0)
    ex, sm = _run_alpha(ktab, qe, src, dst, t2, msg2,
                        W_time.reshape(1, 16), b_time.reshape(1, 16))
    vagg = _run_vagg(vlo, vhi, ex, src, dst)
    return _run_post(sm, vagg, enc, We, row(be), W_asym, row(b_asym))


# speedup vs baseline: 7.8601x; 1.0420x over previous
import jax
import jax.numpy as jnp
from jax import lax
from jax.experimental import pallas as pl
from jax.experimental.pallas import tpu as pltpu
from jax.experimental.pallas import tpu_sc as plsc

N = 10000
E = 320000
MEM = 128
GAMMA = 0.1
EPSILON = 1.0
INV_SQRT_MEM = 1.0 / (128.0 ** 0.5)

K_D = 144
QE_D = 176
SM_D = 48
V_D = 64

NC = 2
NS = 16
NW = NC * NS
N2 = 10240
E2 = 327680
EPW = E2 // NW
C = 80
NCH = EPW // C
G = C // 16
RPT = N2 // NS
CPR = 128
NCP = RPT // CPR
NCH2 = 2 * NCH

BRP = 256
GRIDP = N2 // BRP
BR = 200
GRID = N // BR

TWO_PI = 6.283185307179586
INV_2PI = 1.0 / TWO_PI
COS_COEF = (0.9999999999193508, -19.739208758208584, 64.93939011340913,
            -85.45668538180254, 60.24246470872289, -26.406761080377983,
            7.806608463960106, -1.4609479689305238)


def _cos_polyN(zs):
    us = [z * INV_2PI for z in zs]
    ns = [u.astype(jnp.int32).astype(jnp.float32) for u in us]
    fs = [u - n for u, n in zip(us, ns)]
    fs = [jnp.where(f > 0.5, f - 1.0, f) for f in fs]
    fs = [jnp.where(f < -0.5, f + 1.0, f) for f in fs]
    ys = [f * f for f in fs]
    c = COS_COEF
    y2s = [y * y for y in ys]
    y4s = [y2 * y2 for y2 in y2s]
    p01 = [c[0] + c[1] * y for y in ys]
    p23 = [c[2] + c[3] * y for y in ys]
    p45 = [c[4] + c[5] * y for y in ys]
    p67 = [c[6] + c[7] * y for y in ys]
    return [(a + b * y2) + (d + g * y2) * y4
            for a, b, d, g, y2, y4 in zip(p01, p23, p45, p67, y2s, y4s)]


def _pre_body(x_ref, lu_ref, W_enc_ref, b_enc_ref, Wq_ref, bq_ref, Wk_ref,
              bk_ref, Wv_ref, bv_ref, We_ref, be_col_ref,
              k_ref, vlo_ref, vhi_ref, qe_ref, enc_ref):
    f32 = jnp.float32
    x = x_ref[...]
    dotT = lambda a, w: lax.dot_general(a, w, (((1,), (1,)), ((), ())),
                                        preferred_element_type=f32)
    enc = dotT(x, W_enc_ref[...]) + b_enc_ref[...]
    q = dotT(enc, Wq_ref[...]) + bq_ref[...]
    k = dotT(enc, Wk_ref[...]) + bk_ref[...]
    v = dotT(enc, Wv_ref[...]) + bv_ref[...]
    em = jnp.dot(q, We_ref[...], preferred_element_type=f32)
    qb = jnp.dot(q, be_col_ref[...], preferred_element_type=f32)
    pad = jnp.zeros((BRP, 15), dtype=f32)
    k_ref[...] = jnp.concatenate([k, lu_ref[...], pad], axis=1)
    vlo_ref[...] = v[:, 0:V_D]
    vhi_ref[...] = v[:, V_D:MEM]
    qe_ref[...] = jnp.concatenate([q, em, qb, pad], axis=1)
    enc_ref[...] = enc


def _run_pre(x, lu2, W_enc, b_enc, Wq, bq, Wk, bk, Wv, bv, We, be_col):
    f32 = jnp.float32
    row = lambda d: pl.BlockSpec((BRP, d), lambda i: (i, 0))
    full = lambda a, b: pl.BlockSpec((a, b), lambda i: (0, 0))
    return pl.pallas_call(
        _pre_body,
        grid=(GRIDP,),
        in_specs=[row(MEM), row(1), full(MEM, MEM), full(1, MEM),
                  full(MEM, MEM), full(1, MEM), full(MEM, MEM), full(1, MEM),
                  full(MEM, MEM), full(1, MEM), full(MEM, 32), full(MEM, 1)],
        out_specs=[row(K_D), row(V_D), row(V_D), row(QE_D), row(MEM)],
        out_shape=[jax.ShapeDtypeStruct((N2, K_D), f32),
                   jax.ShapeDtypeStruct((N2, V_D), f32),
                   jax.ShapeDtypeStruct((N2, V_D), f32),
                   jax.ShapeDtypeStruct((N2, QE_D), f32),
                   jax.ShapeDtypeStruct((N2, MEM), f32)],
    )(x, lu2, W_enc, b_enc, Wq, bq, Wk, bk, Wv, bv, We, be_col)


def _alpha_body(k_hbm, qe_hbm, src_hbm, dst_hbm, t_hbm, msg_hbm, wt_hbm,
                bt_hbm, ex_hbm, out_hbm,
                srcv0, dstv0, tv0, msgv0, srcv1, dstv1, tv1, msgv1,
                srcv2, dstv2, tv2, msgv2, srcv3, dstv3, tv3, msgv3,
                kvv0, qev0, kvv1, qev1, outv0, outv1,
                exall, wtv, btv, zb, accum,
                semL0, semL1, semG0, semG1, semS0, semS1):
    c = lax.axis_index("c")
    s = lax.axis_index("s")
    wid = s * NC + c

    srcv = (srcv0, srcv1, srcv2, srcv3)
    dstv = (dstv0, dstv1, dstv2, dstv3)
    tv = (tv0, tv1, tv2, tv3)
    msgv = (msgv0, msgv1, msgv2, msgv3)
    kvv = (kvv0, kvv1)
    qev = (qev0, qev1)
    outv = (outv0, outv1)
    semL = (semL0, semL1)
    semG = (semG0, semG1)
    semS = (semS0, semS1)

    pltpu.sync_copy(wt_hbm, wtv)
    pltpu.sync_copy(bt_hbm, btv)
    wt = wtv[0, pl.ds(0, 16)]
    bt = btv[0, pl.ds(0, 16)]
    lane = lax.iota(jnp.int32, 16)
    unit = jnp.where(lane == 0, 1.0, 0.0).astype(jnp.float32)
    _dn = lax.GatherDimensionNumbers(offset_dims=(), collapsed_slice_dims=(0,),
                                     start_index_map=(0,))
    _perms = [(lane ^ m)[:, None] for m in (8, 4, 2, 1)]

    def _allsumN(vals):
        for p in _perms:
            sh = [lax.gather(a, p, _dn, slice_sizes=(1,),
                             mode=lax.GatherScatterMode.PROMISE_IN_BOUNDS)
                  for a in vals]
            vals = [a + s for a, s in zip(vals, sh)]
        return vals

    def _bcast(v, j):
        idx = jnp.full((16, 1), j, jnp.int32)
        return lax.gather(v, idx, _dn, slice_sizes=(1,),
                          mode=lax.GatherScatterMode.PROMISE_IN_BOUNDS)

    zeros16 = jnp.zeros((16,), jnp.float32)

    def zrow(j, carry):
        for kk in range(SM_D // 16):
            zb[j, pl.ds(kk * 16, 16)] = zeros16
        return carry

    lax.fori_loop(0, CPR, zrow, 0)
    for j in range(NCP):
        pltpu.sync_copy(zb, accum.at[pl.ds(s * RPT + j * CPR, CPR)])
    plsc.subcore_barrier()

    def lin_issue(ci, l4, l2):
        base = jnp.minimum(ci, NCH - 1) * C
        return [
            pltpu.async_copy(src_hbm.at[wid, pl.ds(base, C)], srcv[l4], semL[l2]),
            pltpu.async_copy(dst_hbm.at[wid, pl.ds(base, C)], dstv[l4], semL[l2]),
            pltpu.async_copy(t_hbm.at[wid, pl.ds(base, C)], tv[l4].at[pl.ds(0, C)], semL[l2]),
            pltpu.async_copy(msg_hbm.at[pl.ds(wid * EPW + base, C)], msgv[l4], semL[l2]),
        ]

    def lin_drain(l4, l2):
        pltpu.make_async_copy(src_hbm.at[wid, pl.ds(0, C)], srcv[l4], semL[l2]).wait()
        pltpu.make_async_copy(dst_hbm.at[wid, pl.ds(0, C)], dstv[l4], semL[l2]).wait()
        pltpu.make_async_copy(t_hbm.at[wid, pl.ds(0, C)], tv[l4].at[pl.ds(0, C)], semL[l2]).wait()
        pltpu.make_async_copy(msg_hbm.at[pl.ds(0, C)], msgv[l4], semL[l2]).wait()

    def gat_issue(l4, k2):
        pltpu.async_copy(k_hbm.at[srcv[l4]], kvv[k2], semG[k2])
        pltpu.async_copy(qe_hbm.at[dstv[l4]], qev[k2], semG[k2])

    def gat_drain(l4, k2):
        pltpu.make_async_copy(k_hbm.at[srcv[l4]], kvv[k2], semG[k2]).wait()
        pltpu.make_async_copy(qe_hbm.at[dstv[l4]], qev[k2], semG[k2]).wait()

    def sca_issue(l4, k2):
        pltpu.async_copy(outv[k2], accum.at[dstv[l4]], semS[k2], add=True)

    def sca_drain(l4, k2):
        pltpu.make_async_copy(outv[k2], accum.at[dstv[l4]], semS[k2]).wait()

    lane0 = lane == 0

    def compute(ci, l4, k2):
        kv = kvv[k2]
        qe = qev[k2]
        ov = outv[k2]
        mv = msgv[l4]
        tt = tv[l4]

        def pair(e, exg):
            es = (e, e + 1)
            prods = lambda r: [qe[ei, pl.ds(16 * r, 16)] * kv[ei, pl.ds(16 * r, 16)]
                               for ei in es]
            acc_a = prods(0)
            acc_b = prods(1)
            for r in range(2, 8, 2):
                pa = prods(r)
                pb = prods(r + 1)
                acc_a = [x + y for x, y in zip(acc_a, pa)]
                acc_b = [x + y for x, y in zip(acc_b, pb)]
            accs = [x + y for x, y in zip(acc_a, acc_b)]
            msgs = [mv[ei, pl.ds(0, 16)] for ei in es]
            rels = [jnp.abs(_bcast(kv[ei, pl.ds(128, 16)], 0)
                            - _bcast(tt[pl.ds(ei, 16)], 0)) for ei in es]
            tes = _cos_polyN([r_ * wt + bt for r_ in rels])
            accs = [a + qe[ei, pl.ds(128, 16)] * m
                    for a, ei, m in zip(accs, es, msgs)]
            accs = [a + qe[ei, pl.ds(144, 16)] * t_
                    for a, ei, t_ in zip(accs, es, tes)]
            accs = [a + qe[ei, pl.ds(160, 16)]
                    for a, ei in zip(accs, es)]
            exs = [jnp.exp(s_ * INV_SQRT_MEM) for s_ in _allsumN(accs)]
            for ei, exx, m, t_ in zip(es, exs, msgs, tes):
                ov[ei, pl.ds(0, 16)] = exx * m
                ov[ei, pl.ds(16, 16)] = exx * t_
                ov[ei, pl.ds(32, 16)] = exx * unit
            j = e & 15
            exg = jnp.where(lane == j, exs[0], exg)
            exg = jnp.where(lane == j + 1, exs[1], exg)

            @pl.when(j == 14)
            def _():
                exall[pl.ds(ci * C + e - 14, 16)] = exg

            return exg

        plsc.parallel_loop(0, C, 2, unroll=2, carry=zeros16)(pair)

    lin_issue(0, 0, 0)
    lin_issue(1, 1, 1)
    lin_drain(0, 0)
    gat_issue(0, 0)

    def quad(q4, carry):
        i0 = 4 * q4
        for j in range(4):
            i = i0 + j
            l4 = j
            k2 = j % 2

            @pl.when(i >= 2)
            def _():
                sca_drain((j + 2) % 4, k2)

            la = lin_issue(i + 2, (j + 2) % 4, k2)
            lin_drain((j + 1) % 4, (j + 1) % 2)
            gat_issue((j + 1) % 4, (j + 1) % 2)
            gat_drain(l4, k2)
            compute(i, l4, k2)
            sca_issue(l4, k2)
            del la
        return carry

    lax.fori_loop(0, NCH // 4, quad, 0)
    sca_drain(2, 0)
    sca_drain(3, 1)
    lin_drain(1, 1)
    gat_drain(0, 0)

    pltpu.sync_copy(exall, ex_hbm.at[wid])

    plsc.subcore_barrier()
    for j in range(NCP):
        r0 = s * RPT + j * CPR
        pltpu.sync_copy(accum.at[pl.ds(r0, CPR)], zb)
        pltpu.sync_copy(zb, out_hbm.at[c, pl.ds(r0, CPR)])


def _run_alpha(ktab, qe, src, dst, t, msg, wt, bt):
    f32 = jnp.float32
    mesh = plsc.VectorSubcoreMesh(core_axis_name="c", subcore_axis_name="s",
                                  num_cores=NC, num_subcores=NS)
    fn = pl.kernel(
        _alpha_body,
        out_type=[jax.ShapeDtypeStruct((NW, EPW), f32),
                  jax.ShapeDtypeStruct((NC, N2, SM_D), f32)],
        mesh=mesh,
        compiler_params=pltpu.CompilerParams(use_tc_tiling_on_sc=False),
        scratch_types=(
            4 * [
                pltpu.VMEM((C,), jnp.int32),
                pltpu.VMEM((C,), jnp.int32),
                pltpu.VMEM((C + 16,), f32),
                pltpu.VMEM((C, 16), f32),
            ]
            + 2 * [
                pltpu.VMEM((C, K_D), f32),
                pltpu.VMEM((C, QE_D), f32),
            ]
            + 2 * [
                pltpu.VMEM((C, SM_D), f32),
            ]
            + [
                pltpu.VMEM((EPW,), f32),
                pltpu.VMEM((1, 16), f32),
                pltpu.VMEM((1, 16), f32),
                pltpu.VMEM((CPR, SM_D), f32),
                pltpu.VMEM_SHARED((N2, SM_D), f32),
            ]
            + 6 * [pltpu.SemaphoreType.DMA]
        ),
    )
    return fn(ktab, qe, src, dst, t, msg, wt, bt)


def _vagg_body(vlo_hbm, vhi_hbm, ex_hbm, src_hbm, dst_hbm, out_hbm,
               srcv0, dstv0, exv0, srcv1, dstv1, exv1,
               srcv2, dstv2, exv2, srcv3, dstv3, exv3,
               vv0, vv1, outv0, outv1, zb, accum,
               semL0, semL1, semG0, semG1, semS0, semS1):
    c = lax.axis_index("c")
    s = lax.axis_index("s")

    srcv = (srcv0, srcv1, srcv2, srcv3)
    dstv = (dstv0, dstv1, dstv2, dstv3)
    exv = (exv0, exv1, exv2, exv3)
    vv = (vv0, vv1)
    outv = (outv0, outv1)
    semL = (semL0, semL1)
    semG = (semG0, semG1)
    semS = (semS0, semS1)

    zeros16 = jnp.zeros((16,), jnp.float32)

    def zrow(j, carry):
        for kk in range(V_D // 16):
            zb[j, pl.ds(kk * 16, 16)] = zeros16
        return carry

    lax.fori_loop(0, CPR, zrow, 0)
    for j in range(NCP):
        pltpu.sync_copy(zb, accum.at[pl.ds(s * RPT + j * CPR, CPR)])
    plsc.subcore_barrier()

    def lin_issue(ci, l4, l2):
        cc = jnp.minimum(ci, NCH2 - 1)
        row = s * 2 + cc // NCH
        base = (cc % NCH) * C
        return [
            pltpu.async_copy(src_hbm.at[row, pl.ds(base, C)], srcv[l4], semL[l2]),
            pltpu.async_copy(dst_hbm.at[row, pl.ds(base, C)], dstv[l4], semL[l2]),
            pltpu.async_copy(ex_hbm.at[row, pl.ds(base, C)], exv[l4].at[pl.ds(0, C)], semL[l2]),
        ]

    def lin_drain(l4, l2):
        pltpu.make_async_copy(src_hbm.at[0, pl.ds(0, C)], srcv[l4], semL[l2]).wait()
        pltpu.make_async_copy(dst_hbm.at[0, pl.ds(0, C)], dstv[l4], semL[l2]).wait()
        pltpu.make_async_copy(ex_hbm.at[0, pl.ds(0, C)], exv[l4].at[pl.ds(0, C)], semL[l2]).wait()

    def gat_issue(l4, k2):
        @pl.when(c == 0)
        def _():
            pltpu.async_copy(vlo_hbm.at[srcv[l4]], vv[k2], semG[k2])

        @pl.when(c == 1)
        def _():
            pltpu.async_copy(vhi_hbm.at[srcv[l4]], vv[k2], semG[k2])

    def gat_drain(l4, k2):
        pltpu.make_async_copy(vlo_hbm.at[srcv[l4]], vv[k2], semG[k2]).wait()

    def sca_issue(l4, k2):
        pltpu.async_copy(outv[k2], accum.at[dstv[l4]], semS[k2], add=True)

    def sca_drain(l4, k2):
        pltpu.make_async_copy(outv[k2], accum.at[dstv[l4]], semS[k2]).wait()

    _dn = lax.GatherDimensionNumbers(offset_dims=(), collapsed_slice_dims=(0,),
                                     start_index_map=(0,))

    def _bcast(v, j):
        idx = jnp.full((16, 1), j, jnp.int32)
        return lax.gather(v, idx, _dn, slice_sizes=(1,),
                          mode=lax.GatherScatterMode.PROMISE_IN_BOUNDS)

    def compute(l4, k2):
        ev = exv[l4]
        va = vv[k2]
        ov = outv[k2]

        def pair(e):
            ex0 = _bcast(ev[pl.ds(e, 16)], 0)
            ex1 = _bcast(ev[pl.ds(e + 1, 16)], 0)
            for r in range(V_D // 16):
                ov[e, pl.ds(16 * r, 16)] = ex0 * va[e, pl.ds(16 * r, 16)]
                ov[e + 1, pl.ds(16 * r, 16)] = ex1 * va[e + 1, pl.ds(16 * r, 16)]

        plsc.parallel_loop(0, C, 2, unroll=2)(pair)

    lin_issue(0, 0, 0)
    lin_issue(1, 1, 1)
    lin_drain(0, 0)
    gat_issue(0, 0)

    def quad(q4, carry):
        i0 = 4 * q4
        for j in range(4):
            i = i0 + j
            l4 = j
            k2 = j % 2

            @pl.when(i >= 2)
            def _():
                sca_drain((j + 2) % 4, k2)

            la = lin_issue(i + 2, (j + 2) % 4, k2)
            lin_drain((j + 1) % 4, (j + 1) % 2)
            gat_issue((j + 1) % 4, (j + 1) % 2)
            gat_drain(l4, k2)
            compute(l4, k2)
            sca_issue(l4, k2)
            del la
        return carry

    lax.fori_loop(0, NCH2 // 4, quad, 0)
    sca_drain(2, 0)
    sca_drain(3, 1)
    lin_drain(1, 1)
    gat_drain(0, 0)

    plsc.subcore_barrier()
    for j in range(NCP):
        r0 = s * RPT + j * CPR
        pltpu.sync_copy(accum.at[pl.ds(r0, CPR)], zb)
        pltpu.sync_copy(zb, out_hbm.at[c, pl.ds(r0, CPR)])


def _run_vagg(vlo, vhi, ex, src, dst):
    f32 = jnp.float32
    mesh = plsc.VectorSubcoreMesh(core_axis_name="c", subcore_axis_name="s",
                                  num_cores=NC, num_subcores=NS)
    fn = pl.kernel(
        _vagg_body,
        out_type=jax.ShapeDtypeStruct((NC, N2, V_D), f32),
        mesh=mesh,
        compiler_params=pltpu.CompilerParams(use_tc_tiling_on_sc=False),
        scratch_types=(
            4 * [
                pltpu.VMEM((C,), jnp.int32),
                pltpu.VMEM((C,), jnp.int32),
                pltpu.VMEM((C + 16,), f32),
            ]
            + 2 * [
                pltpu.VMEM((C, V_D), f32),
            ]
            + 2 * [
                pltpu.VMEM((C, V_D), f32),
            ]
            + [
                pltpu.VMEM((CPR, V_D), f32),
                pltpu.VMEM_SHARED((N2, V_D), f32),
            ]
            + 6 * [pltpu.SemaphoreType.DMA]
        ),
    )
    return fn(vlo, vhi, ex, src, dst)


def _post_body(sm_ref, vagg_ref, enc_ref, We_ref, be_row_ref, Wa_ref,
               ba_row_ref, out_ref):
    f32 = jnp.float32
    S = sm_ref[0] + sm_ref[1]
    Sm = S[:, 0:16]
    St = S[:, 16:32]
    Sd = S[:, 32:33]
    Sv = jnp.concatenate([vagg_ref[0], vagg_ref[1]], axis=1)
    We = We_ref[...]
    dotT = lambda a, w: lax.dot_general(a, w, (((1,), (1,)), ((), ())),
                                        preferred_element_type=f32)
    num = Sv + dotT(Sm, We[:, 0:16]) + dotT(St, We[:, 16:32]) \
        + Sd * be_row_ref[...]
    conv = num / (Sd + 1e-16)
    enc = enc_ref[...]
    Wa = Wa_ref[...]
    lin = dotT(enc, Wa) - jnp.dot(enc, Wa, preferred_element_type=f32) \
        - GAMMA * enc
    h = jnp.tanh(lin + conv + ba_row_ref[...])
    out_ref[...] = jnp.tanh(enc + EPSILON * h)


def _run_post(sm, vagg, enc, We, be_row, Wa, ba_row):
    f32 = jnp.float32
    return pl.pallas_call(
        _post_body,
        grid=(GRID,),
        in_specs=[pl.BlockSpec((NC, BR, SM_D), lambda i: (0, i, 0)),
                  pl.BlockSpec((NC, BR, V_D), lambda i: (0, i, 0)),
                  pl.BlockSpec((BR, MEM), lambda i: (i, 0)),
                  pl.BlockSpec((MEM, 32), lambda i: (0, 0)),
                  pl.BlockSpec((1, MEM), lambda i: (0, 0)),
                  pl.BlockSpec((MEM, MEM), lambda i: (0, 0)),
                  pl.BlockSpec((1, MEM), lambda i: (0, 0))],
        out_specs=pl.BlockSpec((BR, MEM), lambda i: (i, 0)),
        out_shape=jax.ShapeDtypeStruct((N, MEM), f32),
    )(sm, vagg, enc, We, be_row, Wa, ba_row)


def kernel(x, last_update, edge_index, t, msg, W_time, b_time, W_enc, b_enc,
           Wq, bq, Wk, bk, Wv, bv, We, be, W_asym, b_asym):
    PN = N2 - N
    PE = E2 - E
    x2 = jnp.concatenate([x, jnp.zeros((PN, MEM), jnp.float32)], axis=0)
    lu2 = jnp.concatenate([last_update, jnp.zeros((PN,), jnp.float32)]
                          ).reshape(N2, 1)
    row = lambda b: b.reshape(1, MEM)
    ktab, vlo, vhi, qe, enc = _run_pre(x2, lu2, W_enc, row(b_enc), Wq, row(bq),
                                       Wk, row(bk), Wv, row(bv), We,
                                       be.reshape(MEM, 1))
    src = jnp.concatenate([edge_index[0], jnp.zeros((PE,), jnp.int32)]
                          ).reshape(NW, EPW)
    dst = jnp.concatenate([edge_index[1], jnp.full((PE,), N2 - 1, jnp.int32)]
                          ).reshape(NW, EPW)
    t2 = jnp.concatenate([t, jnp.zeros((PE,), jnp.float32)]).reshape(NW, EPW)
    msg2 = jnp.concatenate([msg, jnp.zeros((PE, 16), jnp.float32)], axis=0)
    ex, sm = _run_alpha(ktab, qe, src, dst, t2, msg2,
                        W_time.reshape(1, 16), b_time.reshape(1, 16))
    vagg = _run_vagg(vlo, vhi, ex, src, dst)
    return _run_post(sm, vagg, enc, We, row(be), W_asym, row(b_asym))


# speedup vs baseline: 7.9000x; 1.0051x over previous
import jax
import jax.numpy as jnp
from jax import lax
from jax.experimental import pallas as pl
from jax.experimental.pallas import tpu as pltpu
from jax.experimental.pallas import tpu_sc as plsc

N = 10000
E = 320000
MEM = 128
GAMMA = 0.1
EPSILON = 1.0
INV_SQRT_MEM = 1.0 / (128.0 ** 0.5)

K_D = 144
QE_D = 176
SM_D = 48
V_D = 64

NC = 2
NS = 16
NW = NC * NS
N2 = 10240
E2 = 327680
EPW = E2 // NW
C = 80
NCH = EPW // C
G = C // 16
RPT = N2 // NS
CPR = 128
NCP = RPT // CPR
NCH2 = 2 * NCH

BRP = 256
GRIDP = N2 // BRP
BR = 200
GRID = N // BR

TWO_PI = 6.283185307179586
INV_2PI = 1.0 / TWO_PI
COS_COEF = (0.9999999999193508, -19.739208758208584, 64.93939011340913,
            -85.45668538180254, 60.24246470872289, -26.406761080377983,
            7.806608463960106, -1.4609479689305238)


def _cos_polyN(zs):
    us = [z * INV_2PI for z in zs]
    ns = [u.astype(jnp.int32).astype(jnp.float32) for u in us]
    fs = [u - n for u, n in zip(us, ns)]
    fs = [jnp.where(f > 0.5, f - 1.0, f) for f in fs]
    fs = [jnp.where(f < -0.5, f + 1.0, f) for f in fs]
    ys = [f * f for f in fs]
    c = COS_COEF
    y2s = [y * y for y in ys]
    y4s = [y2 * y2 for y2 in y2s]
    p01 = [c[0] + c[1] * y for y in ys]
    p23 = [c[2] + c[3] * y for y in ys]
    p45 = [c[4] + c[5] * y for y in ys]
    p67 = [c[6] + c[7] * y for y in ys]
    return [(a + b * y2) + (d + g * y2) * y4
            for a, b, d, g, y2, y4 in zip(p01, p23, p45, p67, y2s, y4s)]


def _pre_body(x_ref, lu_ref, W_enc_ref, b_enc_ref, Wq_ref, bq_ref, Wk_ref,
              bk_ref, Wv_ref, bv_ref, We_ref, be_col_ref,
              k_ref, vlo_ref, vhi_ref, qe_ref, enc_ref):
    f32 = jnp.float32
    x = x_ref[...]
    dotT = lambda a, w: lax.dot_general(a, w, (((1,), (1,)), ((), ())),
                                        preferred_element_type=f32)
    enc = dotT(x, W_enc_ref[...]) + b_enc_ref[...]
    q = dotT(enc, Wq_ref[...]) + bq_ref[...]
    k = dotT(enc, Wk_ref[...]) + bk_ref[...]
    v = dotT(enc, Wv_ref[...]) + bv_ref[...]
    em = jnp.dot(q, We_ref[...], preferred_element_type=f32)
    qb = jnp.dot(q, be_col_ref[...], preferred_element_type=f32)
    pad = jnp.zeros((BRP, 15), dtype=f32)
    k_ref[...] = jnp.concatenate([k, lu_ref[...], pad], axis=1)
    vlo_ref[...] = v[:, 0:V_D]
    vhi_ref[...] = v[:, V_D:MEM]
    qe_ref[...] = jnp.concatenate([q, em, qb, pad], axis=1)
    enc_ref[...] = enc


def _run_pre(x, lu2, W_enc, b_enc, Wq, bq, Wk, bk, Wv, bv, We, be_col):
    f32 = jnp.float32
    row = lambda d: pl.BlockSpec((BRP, d), lambda i: (i, 0))
    full = lambda a, b: pl.BlockSpec((a, b), lambda i: (0, 0))
    return pl.pallas_call(
        _pre_body,
        grid=(GRIDP,),
        in_specs=[row(MEM), row(1), full(MEM, MEM), full(1, MEM),
                  full(MEM, MEM), full(1, MEM), full(MEM, MEM), full(1, MEM),
                  full(MEM, MEM), full(1, MEM), full(MEM, 32), full(MEM, 1)],
        out_specs=[row(K_D), row(V_D), row(V_D), row(QE_D), row(MEM)],
        out_shape=[jax.ShapeDtypeStruct((N2, K_D), f32),
                   jax.ShapeDtypeStruct((N2, V_D), f32),
                   jax.ShapeDtypeStruct((N2, V_D), f32),
                   jax.ShapeDtypeStruct((N2, QE_D), f32),
                   jax.ShapeDtypeStruct((N2, MEM), f32)],
    )(x, lu2, W_enc, b_enc, Wq, bq, Wk, bk, Wv, bv, We, be_col)


def _alpha_body(k_hbm, qe_hbm, src_hbm, dst_hbm, t_hbm, msg_hbm, wt_hbm,
                bt_hbm, ex_hbm, out_hbm,
                srcv0, dstv0, tv0, msgv0, srcv1, dstv1, tv1, msgv1,
                srcv2, dstv2, tv2, msgv2, srcv3, dstv3, tv3, msgv3,
                kvv0, qev0, kvv1, qev1, outv0, outv1,
                exall, wtv, btv, zb, accum,
                semL0, semL1, semG0, semG1, semS0, semS1):
    c = lax.axis_index("c")
    s = lax.axis_index("s")
    wid = s * NC + c

    srcv = (srcv0, srcv1, srcv2, srcv3)
    dstv = (dstv0, dstv1, dstv2, dstv3)
    tv = (tv0, tv1, tv2, tv3)
    msgv = (msgv0, msgv1, msgv2, msgv3)
    kvv = (kvv0, kvv1)
    qev = (qev0, qev1)
    outv = (outv0, outv1)
    semL = (semL0, semL1)
    semG = (semG0, semG1)
    semS = (semS0, semS1)

    pltpu.sync_copy(wt_hbm, wtv)
    pltpu.sync_copy(bt_hbm, btv)
    wt = wtv[0, pl.ds(0, 16)]
    bt = btv[0, pl.ds(0, 16)]
    lane = lax.iota(jnp.int32, 16)
    unit = jnp.where(lane == 0, 1.0, 0.0).astype(jnp.float32)
    _dn = lax.GatherDimensionNumbers(offset_dims=(), collapsed_slice_dims=(0,),
                                     start_index_map=(0,))
    _perms = [(lane ^ m)[:, None] for m in (8, 4, 2, 1)]

    def _allsumN(vals):
        for p in _perms:
            sh = [lax.gather(a, p, _dn, slice_sizes=(1,),
                             mode=lax.GatherScatterMode.PROMISE_IN_BOUNDS)
                  for a in vals]
            vals = [a + s for a, s in zip(vals, sh)]
        return vals

    def _bcast(v, j):
        idx = jnp.full((16, 1), j, jnp.int32)
        return lax.gather(v, idx, _dn, slice_sizes=(1,),
                          mode=lax.GatherScatterMode.PROMISE_IN_BOUNDS)

    zeros16 = jnp.zeros((16,), jnp.float32)

    def zrow(j, carry):
        for kk in range(SM_D // 16):
            zb[j, pl.ds(kk * 16, 16)] = zeros16
        return carry

    lax.fori_loop(0, CPR, zrow, 0)
    for j in range(NCP):
        pltpu.sync_copy(zb, accum.at[pl.ds(s * RPT + j * CPR, CPR)])
    plsc.subcore_barrier()

    def lin_issue(ci, l4, l2):
        base = jnp.minimum(ci, NCH - 1) * C
        return [
            pltpu.async_copy(src_hbm.at[wid, pl.ds(base, C)], srcv[l4], semL[l2]),
            pltpu.async_copy(dst_hbm.at[wid, pl.ds(base, C)], dstv[l4], semL[l2]),
            pltpu.async_copy(t_hbm.at[wid, pl.ds(base, C)], tv[l4].at[pl.ds(0, C)], semL[l2]),
            pltpu.async_copy(msg_hbm.at[pl.ds(wid * EPW + base, C)], msgv[l4], semL[l2]),
        ]

    def lin_drain(l4, l2):
        pltpu.make_async_copy(src_hbm.at[wid, pl.ds(0, C)], srcv[l4], semL[l2]).wait()
        pltpu.make_async_copy(dst_hbm.at[wid, pl.ds(0, C)], dstv[l4], semL[l2]).wait()
        pltpu.make_async_copy(t_hbm.at[wid, pl.ds(0, C)], tv[l4].at[pl.ds(0, C)], semL[l2]).wait()
        pltpu.make_async_copy(msg_hbm.at[pl.ds(0, C)], msgv[l4], semL[l2]).wait()

    def gat_issue(l4, k2):
        pltpu.async_copy(k_hbm.at[srcv[l4]], kvv[k2], semG[k2])
        pltpu.async_copy(qe_hbm.at[dstv[l4]], qev[k2], semG[k2])

    def gat_drain(l4, k2):
        pltpu.make_async_copy(k_hbm.at[srcv[l4]], kvv[k2], semG[k2]).wait()
        pltpu.make_async_copy(qe_hbm.at[dstv[l4]], qev[k2], semG[k2]).wait()

    def sca_issue(l4, k2):
        pltpu.async_copy(outv[k2], accum.at[dstv[l4]], semS[k2], add=True)

    def sca_drain(l4, k2):
        pltpu.make_async_copy(outv[k2], accum.at[dstv[l4]], semS[k2]).wait()

    lane0 = lane == 0

    def compute(ci, l4, k2):
        kv = kvv[k2]
        qe = qev[k2]
        ov = outv[k2]
        mv = msgv[l4]
        tt = tv[l4]

        def pair(e, exg):
            es = (e, e + 1, e + 2, e + 3)
            prods = lambda r: [qe[ei, pl.ds(16 * r, 16)] * kv[ei, pl.ds(16 * r, 16)]
                               for ei in es]
            acc_a = prods(0)
            acc_b = prods(1)
            for r in range(2, 8, 2):
                pa = prods(r)
                pb = prods(r + 1)
                acc_a = [x + y for x, y in zip(acc_a, pa)]
                acc_b = [x + y for x, y in zip(acc_b, pb)]
            accs = [x + y for x, y in zip(acc_a, acc_b)]
            msgs = [mv[ei, pl.ds(0, 16)] for ei in es]
            rels = [jnp.abs(_bcast(kv[ei, pl.ds(128, 16)], 0)
                            - _bcast(tt[pl.ds(ei, 16)], 0)) for ei in es]
            tes = _cos_polyN([r_ * wt + bt for r_ in rels])
            accs = [a + qe[ei, pl.ds(128, 16)] * m
                    for a, ei, m in zip(accs, es, msgs)]
            accs = [a + qe[ei, pl.ds(144, 16)] * t_
                    for a, ei, t_ in zip(accs, es, tes)]
            accs = [a + qe[ei, pl.ds(160, 16)]
                    for a, ei in zip(accs, es)]
            exs = [jnp.exp(s_ * INV_SQRT_MEM) for s_ in _allsumN(accs)]
            for ei, exx, m, t_ in zip(es, exs, msgs, tes):
                ov[ei, pl.ds(0, 16)] = exx * m
                ov[ei, pl.ds(16, 16)] = exx * t_
                ov[ei, pl.ds(32, 16)] = exx * unit
            j = e & 15
            for jj in range(4):
                exg = jnp.where(lane == j + jj, exs[jj], exg)

            @pl.when(j == 12)
            def _():
                exall[pl.ds(ci * C + e - 12, 16)] = exg

            return exg

        plsc.parallel_loop(0, C, 4, unroll=1, carry=zeros16)(pair)

    lin_issue(0, 0, 0)
    lin_issue(1, 1, 1)
    lin_drain(0, 0)
    gat_issue(0, 0)

    def quad(q4, carry):
        i0 = 4 * q4
        for j in range(4):
            i = i0 + j
            l4 = j
            k2 = j % 2

            @pl.when(i >= 2)
            def _():
                sca_drain((j + 2) % 4, k2)

            la = lin_issue(i + 2, (j + 2) % 4, k2)
            lin_drain((j + 1) % 4, (j + 1) % 2)
            gat_issue((j + 1) % 4, (j + 1) % 2)
            gat_drain(l4, k2)
            compute(i, l4, k2)
            sca_issue(l4, k2)
            del la
        return carry

    lax.fori_loop(0, NCH // 4, quad, 0)
    sca_drain(2, 0)
    sca_drain(3, 1)
    lin_drain(1, 1)
    gat_drain(0, 0)

    pltpu.sync_copy(exall, ex_hbm.at[wid])

    plsc.subcore_barrier()
    for j in range(NCP):
        r0 = s * RPT + j * CPR
        pltpu.sync_copy(accum.at[pl.ds(r0, CPR)], zb)
        pltpu.sync_copy(zb, out_hbm.at[c, pl.ds(r0, CPR)])


def _run_alpha(ktab, qe, src, dst, t, msg, wt, bt):
    f32 = jnp.float32
    mesh = plsc.VectorSubcoreMesh(core_axis_name="c", subcore_axis_name="s",
                                  num_cores=NC, num_subcores=NS)
    fn = pl.kernel(
        _alpha_body,
        out_type=[jax.ShapeDtypeStruct((NW, EPW), f32),
                  jax.ShapeDtypeStruct((NC, N2, SM_D), f32)],
        mesh=mesh,
        compiler_params=pltpu.CompilerParams(use_tc_tiling_on_sc=False),
        scratch_types=(
            4 * [
                pltpu.VMEM((C,), jnp.int32),
                pltpu.VMEM((C,), jnp.int32),
                pltpu.VMEM((C + 16,), f32),
                pltpu.VMEM((C, 16), f32),
            ]
            + 2 * [
                pltpu.VMEM((C, K_D), f32),
                pltpu.VMEM((C, QE_D), f32),
            ]
            + 2 * [
                pltpu.VMEM((C, SM_D), f32),
            ]
            + [
                pltpu.VMEM((EPW,), f32),
                pltpu.VMEM((1, 16), f32),
                pltpu.VMEM((1, 16), f32),
                pltpu.VMEM((CPR, SM_D), f32),
                pltpu.VMEM_SHARED((N2, SM_D), f32),
            ]
            + 6 * [pltpu.SemaphoreType.DMA]
        ),
    )
    return fn(ktab, qe, src, dst, t, msg, wt, bt)


def _vagg_body(vlo_hbm, vhi_hbm, ex_hbm, src_hbm, dst_hbm, out_hbm,
               srcv0, dstv0, exv0, srcv1, dstv1, exv1,
               srcv2, dstv2, exv2, srcv3, dstv3, exv3,
               vv0, vv1, outv0, outv1, zb, accum,
               semL0, semL1, semG0, semG1, semS0, semS1):
    c = lax.axis_index("c")
    s = lax.axis_index("s")

    srcv = (srcv0, srcv1, srcv2, srcv3)
    dstv = (dstv0, dstv1, dstv2, dstv3)
    exv = (exv0, exv1, exv2, exv3)
    vv = (vv0, vv1)
    outv = (outv0, outv1)
    semL = (semL0, semL1)
    semG = (semG0, semG1)
    semS = (semS0, semS1)

    zeros16 = jnp.zeros((16,), jnp.float32)

    def zrow(j, carry):
        for kk in range(V_D // 16):
            zb[j, pl.ds(kk * 16, 16)] = zeros16
        return carry

    lax.fori_loop(0, CPR, zrow, 0)
    for j in range(NCP):
        pltpu.sync_copy(zb, accum.at[pl.ds(s * RPT + j * CPR, CPR)])
    plsc.subcore_barrier()

    def lin_issue(ci, l4, l2):
        cc = jnp.minimum(ci, NCH2 - 1)
        row = s * 2 + cc // NCH
        base = (cc % NCH) * C
        return [
            pltpu.async_copy(src_hbm.at[row, pl.ds(base, C)], srcv[l4], semL[l2]),
            pltpu.async_copy(dst_hbm.at[row, pl.ds(base, C)], dstv[l4], semL[l2]),
            pltpu.async_copy(ex_hbm.at[row, pl.ds(base, C)], exv[l4].at[pl.ds(0, C)], semL[l2]),
        ]

    def lin_drain(l4, l2):
        pltpu.make_async_copy(src_hbm.at[0, pl.ds(0, C)], srcv[l4], semL[l2]).wait()
        pltpu.make_async_copy(dst_hbm.at[0, pl.ds(0, C)], dstv[l4], semL[l2]).wait()
        pltpu.make_async_copy(ex_hbm.at[0, pl.ds(0, C)], exv[l4].at[pl.ds(0, C)], semL[l2]).wait()

    def gat_issue(l4, k2):
        @pl.when(c == 0)
        def _():
            pltpu.async_copy(vlo_hbm.at[srcv[l4]], vv[k2], semG[k2])

        @pl.when(c == 1)
        def _():
            pltpu.async_copy(vhi_hbm.at[srcv[l4]], vv[k2], semG[k2])

    def gat_drain(l4, k2):
        pltpu.make_async_copy(vlo_hbm.at[srcv[l4]], vv[k2], semG[k2]).wait()

    def sca_issue(l4, k2):
        pltpu.async_copy(outv[k2], accum.at[dstv[l4]], semS[k2], add=True)

    def sca_drain(l4, k2):
        pltpu.make_async_copy(outv[k2], accum.at[dstv[l4]], semS[k2]).wait()

    _dn = lax.GatherDimensionNumbers(offset_dims=(), collapsed_slice_dims=(0,),
                                     start_index_map=(0,))

    def _bcast(v, j):
        idx = jnp.full((16, 1), j, jnp.int32)
        return lax.gather(v, idx, _dn, slice_sizes=(1,),
                          mode=lax.GatherScatterMode.PROMISE_IN_BOUNDS)

    def compute(l4, k2):
        ev = exv[l4]
        va = vv[k2]
        ov = outv[k2]

        def pair(e):
            ex0 = _bcast(ev[pl.ds(e, 16)], 0)
            ex1 = _bcast(ev[pl.ds(e + 1, 16)], 0)
            for r in range(V_D // 16):
                ov[e, pl.ds(16 * r, 16)] = ex0 * va[e, pl.ds(16 * r, 16)]
                ov[e + 1, pl.ds(16 * r, 16)] = ex1 * va[e + 1, pl.ds(16 * r, 16)]

        plsc.parallel_loop(0, C, 2, unroll=2)(pair)

    lin_issue(0, 0, 0)
    lin_issue(1, 1, 1)
    lin_drain(0, 0)
    gat_issue(0, 0)

    def quad(q4, carry):
        i0 = 4 * q4
        for j in range(4):
            i = i0 + j
            l4 = j
            k2 = j % 2

            @pl.when(i >= 2)
            def _():
                sca_drain((j + 2) % 4, k2)

            la = lin_issue(i + 2, (j + 2) % 4, k2)
            lin_drain((j + 1) % 4, (j + 1) % 2)
            gat_issue((j + 1) % 4, (j + 1) % 2)
            gat_drain(l4, k2)
            compute(l4, k2)
            sca_issue(l4, k2)
            del la
        return carry

    lax.fori_loop(0, NCH2 // 4, quad, 0)
    sca_drain(2, 0)
    sca_drain(3, 1)
    lin_drain(1, 1)
    gat_drain(0, 0)

    plsc.subcore_barrier()
    for j in range(NCP):
        r0 = s * RPT + j * CPR
        pltpu.sync_copy(accum.at[pl.ds(r0, CPR)], zb)
        pltpu.sync_copy(zb, out_hbm.at[c, pl.ds(r0, CPR)])


def _run_vagg(vlo, vhi, ex, src, dst):
    f32 = jnp.float32
    mesh = plsc.VectorSubcoreMesh(core_axis_name="c", subcore_axis_name="s",
                                  num_cores=NC, num_subcores=NS)
    fn = pl.kernel(
        _vagg_body,
        out_type=jax.ShapeDtypeStruct((NC, N2, V_D), f32),
        mesh=mesh,
        compiler_params=pltpu.CompilerParams(use_tc_tiling_on_sc=False),
        scratch_types=(
            4 * [
                pltpu.VMEM((C,), jnp.int32),
                pltpu.VMEM((C,), jnp.int32),
                pltpu.VMEM((C + 16,), f32),
            ]
            + 2 * [
                pltpu.VMEM((C, V_D), f32),
            ]
            + 2 * [
                pltpu.VMEM((C, V_D), f32),
            ]
            + [
                pltpu.VMEM((CPR, V_D), f32),
                pltpu.VMEM_SHARED((N2, V_D), f32),
            ]
            + 6 * [pltpu.SemaphoreType.DMA]
        ),
    )
    return fn(vlo, vhi, ex, src, dst)


def _post_body(sm_ref, vagg_ref, enc_ref, We_ref, be_row_ref, Wa_ref,
               ba_row_ref, out_ref):
    f32 = jnp.float32
    S = sm_ref[0] + sm_ref[1]
    Sm = S[:, 0:16]
    St = S[:, 16:32]
    Sd = S[:, 32:33]
    Sv = jnp.concatenate([vagg_ref[0], vagg_ref[1]], axis=1)
    We = We_ref[...]
    dotT = lambda a, w: lax.dot_general(a, w, (((1,), (1,)), ((), ())),
                                        preferred_element_type=f32)
    num = Sv + dotT(Sm, We[:, 0:16]) + dotT(St, We[:, 16:32]) \
        + Sd * be_row_ref[...]
    conv = num / (Sd + 1e-16)
    enc = enc_ref[...]
    Wa = Wa_ref[...]
    lin = dotT(enc, Wa) - jnp.dot(enc, Wa, preferred_element_type=f32) \
        - GAMMA * enc
    h = jnp.tanh(lin + conv + ba_row_ref[...])
    out_ref[...] = jnp.tanh(enc + EPSILON * h)


def _run_post(sm, vagg, enc, We, be_row, Wa, ba_row):
    f32 = jnp.float32
    return pl.pallas_call(
        _post_body,
        grid=(GRID,),
        in_specs=[pl.BlockSpec((NC, BR, SM_D), lambda i: (0, i, 0)),
                  pl.BlockSpec((NC, BR, V_D), lambda i: (0, i, 0)),
                  pl.BlockSpec((BR, MEM), lambda i: (i, 0)),
                  pl.BlockSpec((MEM, 32), lambda i: (0, 0)),
                  pl.BlockSpec((1, MEM), lambda i: (0, 0)),
                  pl.BlockSpec((MEM, MEM), lambda i: (0, 0)),
                  pl.BlockSpec((1, MEM), lambda i: (0, 0))],
        out_specs=pl.BlockSpec((BR, MEM), lambda i: (i, 0)),
        out_shape=jax.ShapeDtypeStruct((N, MEM), f32),
    )(sm, vagg, enc, We, be_row, Wa, ba_row)


def kernel(x, last_update, edge_index, t, msg, W_time, b_time, W_enc, b_enc,
           Wq, bq, Wk, bk, Wv, bv, We, be, W_asym, b_asym):
    PN = N2 - N
    PE = E2 - E
    x2 = jnp.concatenate([x, jnp.zeros((PN, MEM), jnp.float32)], axis=0)
    lu2 = jnp.concatenate([last_update, jnp.zeros((PN,), jnp.float32)]
                          ).reshape(N2, 1)
    row = lambda b: b.reshape(1, MEM)
    ktab, vlo, vhi, qe, enc = _run_pre(x2, lu2, W_enc, row(b_enc), Wq, row(bq),
                                       Wk, row(bk), Wv, row(bv), We,
                                       be.reshape(MEM, 1))
    src = jnp.concatenate([edge_index[0], jnp.zeros((PE,), jnp.int32)]
                          ).reshape(NW, EPW)
    dst = jnp.concatenate([edge_index[1], jnp.full((PE,), N2 - 1, jnp.int32)]
                          ).reshape(NW, EPW)
    t2 = jnp.concatenate([t, jnp.zeros((PE,), jnp.float32)]).reshape(NW, EPW)
    msg2 = jnp.concatenate([msg, jnp.zeros((PE, 16), jnp.float32)], axis=0)
    ex, sm = _run_alpha(ktab, qe, src, dst, t2, msg2,
                        W_time.reshape(1, 16), b_time.reshape(1, 16))
    vagg = _run_vagg(vlo, vhi, ex, src, dst)
    return _run_post(sm, vagg, enc, We, row(be), W_asym, row(b_asym))


# speedup vs baseline: 7.9798x; 1.0101x over previous
import jax
import jax.numpy as jnp
from jax import lax
from jax.experimental import pallas as pl
from jax.experimental.pallas import tpu as pltpu
from jax.experimental.pallas import tpu_sc as plsc

N = 10000
E = 320000
MEM = 128
GAMMA = 0.1
EPSILON = 1.0
INV_SQRT_MEM = 1.0 / (128.0 ** 0.5)

K_D = 144
QE_D = 176
SM_D = 48
V_D = 64

NC = 2
NS = 16
NW = NC * NS
N2 = 10240
E2 = 327680
EPW = E2 // NW
C = 80
NCH = EPW // C
G = C // 16
RPT = N2 // NS
CPR = 128
NCP = RPT // CPR
NCH2 = 2 * NCH
C2 = 128
NCHV = EPW // C2
NCH2V = 2 * NCHV

BRP = 256
GRIDP = N2 // BRP
BR = 200
GRID = N // BR

TWO_PI = 6.283185307179586
INV_2PI = 1.0 / TWO_PI
COS_COEF = (0.9999999999193508, -19.739208758208584, 64.93939011340913,
            -85.45668538180254, 60.24246470872289, -26.406761080377983,
            7.806608463960106, -1.4609479689305238)


def _cos_polyN(zs):
    us = [z * INV_2PI for z in zs]
    ns = [u.astype(jnp.int32).astype(jnp.float32) for u in us]
    fs = [u - n for u, n in zip(us, ns)]
    fs = [jnp.where(f > 0.5, f - 1.0, f) for f in fs]
    fs = [jnp.where(f < -0.5, f + 1.0, f) for f in fs]
    ys = [f * f for f in fs]
    c = COS_COEF
    y2s = [y * y for y in ys]
    y4s = [y2 * y2 for y2 in y2s]
    p01 = [c[0] + c[1] * y for y in ys]
    p23 = [c[2] + c[3] * y for y in ys]
    p45 = [c[4] + c[5] * y for y in ys]
    p67 = [c[6] + c[7] * y for y in ys]
    return [(a + b * y2) + (d + g * y2) * y4
            for a, b, d, g, y2, y4 in zip(p01, p23, p45, p67, y2s, y4s)]


def _pre_body(x_ref, lu_ref, W_enc_ref, b_enc_ref, Wq_ref, bq_ref, Wk_ref,
              bk_ref, Wv_ref, bv_ref, We_ref, be_col_ref,
              k_ref, vlo_ref, vhi_ref, qe_ref, enc_ref):
    f32 = jnp.float32
    x = x_ref[...]
    dotT = lambda a, w: lax.dot_general(a, w, (((1,), (1,)), ((), ())),
                                        preferred_element_type=f32)
    enc = dotT(x, W_enc_ref[...]) + b_enc_ref[...]
    q = dotT(enc, Wq_ref[...]) + bq_ref[...]
    k = dotT(enc, Wk_ref[...]) + bk_ref[...]
    v = dotT(enc, Wv_ref[...]) + bv_ref[...]
    em = jnp.dot(q, We_ref[...], preferred_element_type=f32)
    qb = jnp.dot(q, be_col_ref[...], preferred_element_type=f32)
    pad = jnp.zeros((BRP, 15), dtype=f32)
    k_ref[...] = jnp.concatenate([k, lu_ref[...], pad], axis=1)
    vlo_ref[...] = v[:, 0:V_D]
    vhi_ref[...] = v[:, V_D:MEM]
    qe_ref[...] = jnp.concatenate([q, em, qb, pad], axis=1)
    enc_ref[...] = enc


def _run_pre(x, lu2, W_enc, b_enc, Wq, bq, Wk, bk, Wv, bv, We, be_col):
    f32 = jnp.float32
    row = lambda d: pl.BlockSpec((BRP, d), lambda i: (i, 0))
    full = lambda a, b: pl.BlockSpec((a, b), lambda i: (0, 0))
    return pl.pallas_call(
        _pre_body,
        grid=(GRIDP,),
        in_specs=[row(MEM), row(1), full(MEM, MEM), full(1, MEM),
                  full(MEM, MEM), full(1, MEM), full(MEM, MEM), full(1, MEM),
                  full(MEM, MEM), full(1, MEM), full(MEM, 32), full(MEM, 1)],
        out_specs=[row(K_D), row(V_D), row(V_D), row(QE_D), row(MEM)],
        out_shape=[jax.ShapeDtypeStruct((N2, K_D), f32),
                   jax.ShapeDtypeStruct((N2, V_D), f32),
                   jax.ShapeDtypeStruct((N2, V_D), f32),
                   jax.ShapeDtypeStruct((N2, QE_D), f32),
                   jax.ShapeDtypeStruct((N2, MEM), f32)],
    )(x, lu2, W_enc, b_enc, Wq, bq, Wk, bk, Wv, bv, We, be_col)


def _alpha_body(k_hbm, qe_hbm, src_hbm, dst_hbm, t_hbm, msg_hbm, wt_hbm,
                bt_hbm, ex_hbm, out_hbm,
                srcv0, dstv0, tv0, msgv0, srcv1, dstv1, tv1, msgv1,
                srcv2, dstv2, tv2, msgv2, srcv3, dstv3, tv3, msgv3,
                kvv0, qev0, kvv1, qev1, outv0, outv1,
                exall, wtv, btv, zb, accum,
                semL0, semL1, semG0, semG1, semS0, semS1):
    c = lax.axis_index("c")
    s = lax.axis_index("s")
    wid = s * NC + c

    srcv = (srcv0, srcv1, srcv2, srcv3)
    dstv = (dstv0, dstv1, dstv2, dstv3)
    tv = (tv0, tv1, tv2, tv3)
    msgv = (msgv0, msgv1, msgv2, msgv3)
    kvv = (kvv0, kvv1)
    qev = (qev0, qev1)
    outv = (outv0, outv1)
    semL = (semL0, semL1)
    semG = (semG0, semG1)
    semS = (semS0, semS1)

    pltpu.sync_copy(wt_hbm, wtv)
    pltpu.sync_copy(bt_hbm, btv)
    wt = wtv[0, pl.ds(0, 16)]
    bt = btv[0, pl.ds(0, 16)]
    lane = lax.iota(jnp.int32, 16)
    unit = jnp.where(lane == 0, 1.0, 0.0).astype(jnp.float32)
    _dn = lax.GatherDimensionNumbers(offset_dims=(), collapsed_slice_dims=(0,),
                                     start_index_map=(0,))
    _perms = [(lane ^ m)[:, None] for m in (8, 4, 2, 1)]

    def _allsumN(vals):
        for p in _perms:
            sh = [lax.gather(a, p, _dn, slice_sizes=(1,),
                             mode=lax.GatherScatterMode.PROMISE_IN_BOUNDS)
                  for a in vals]
            vals = [a + s for a, s in zip(vals, sh)]
        return vals

    def _bcast(v, j):
        idx = jnp.full((16, 1), j, jnp.int32)
        return lax.gather(v, idx, _dn, slice_sizes=(1,),
                          mode=lax.GatherScatterMode.PROMISE_IN_BOUNDS)

    zeros16 = jnp.zeros((16,), jnp.float32)

    def zrow(j, carry):
        for kk in range(SM_D // 16):
            zb[j, pl.ds(kk * 16, 16)] = zeros16
        return carry

    lax.fori_loop(0, CPR, zrow, 0)
    for j in range(NCP):
        pltpu.sync_copy(zb, accum.at[pl.ds(s * RPT + j * CPR, CPR)])
    plsc.subcore_barrier()

    def lin_issue(ci, l4, l2):
        base = jnp.minimum(ci, NCH - 1) * C
        return [
            pltpu.async_copy(src_hbm.at[wid, pl.ds(base, C)], srcv[l4], semL[l2]),
            pltpu.async_copy(dst_hbm.at[wid, pl.ds(base, C)], dstv[l4], semL[l2]),
            pltpu.async_copy(t_hbm.at[wid, pl.ds(base, C)], tv[l4].at[pl.ds(0, C)], semL[l2]),
            pltpu.async_copy(msg_hbm.at[pl.ds(wid * EPW + base, C)], msgv[l4], semL[l2]),
        ]

    def lin_drain(l4, l2):
        pltpu.make_async_copy(src_hbm.at[wid, pl.ds(0, C)], srcv[l4], semL[l2]).wait()
        pltpu.make_async_copy(dst_hbm.at[wid, pl.ds(0, C)], dstv[l4], semL[l2]).wait()
        pltpu.make_async_copy(t_hbm.at[wid, pl.ds(0, C)], tv[l4].at[pl.ds(0, C)], semL[l2]).wait()
        pltpu.make_async_copy(msg_hbm.at[pl.ds(0, C)], msgv[l4], semL[l2]).wait()

    def gat_issue(l4, k2):
        pltpu.async_copy(k_hbm.at[srcv[l4]], kvv[k2], semG[k2])
        pltpu.async_copy(qe_hbm.at[dstv[l4]], qev[k2], semG[k2])

    def gat_drain(l4, k2):
        pltpu.make_async_copy(k_hbm.at[srcv[l4]], kvv[k2], semG[k2]).wait()
        pltpu.make_async_copy(qe_hbm.at[dstv[l4]], qev[k2], semG[k2]).wait()

    def sca_issue(l4, k2):
        pltpu.async_copy(outv[k2], accum.at[dstv[l4]], semS[k2], add=True)

    def sca_drain(l4, k2):
        pltpu.make_async_copy(outv[k2], accum.at[dstv[l4]], semS[k2]).wait()

    lane0 = lane == 0

    def compute(ci, l4, k2):
        kv = kvv[k2]
        qe = qev[k2]
        ov = outv[k2]
        mv = msgv[l4]
        tt = tv[l4]

        def pair(e, exg):
            es = (e, e + 1, e + 2, e + 3)
            prods = lambda r: [qe[ei, pl.ds(16 * r, 16)] * kv[ei, pl.ds(16 * r, 16)]
                               for ei in es]
            acc_a = prods(0)
            acc_b = prods(1)
            for r in range(2, 8, 2):
                pa = prods(r)
                pb = prods(r + 1)
                acc_a = [x + y for x, y in zip(acc_a, pa)]
                acc_b = [x + y for x, y in zip(acc_b, pb)]
            accs = [x + y for x, y in zip(acc_a, acc_b)]
            msgs = [mv[ei, pl.ds(0, 16)] for ei in es]
            rels = [jnp.abs(_bcast(kv[ei, pl.ds(128, 16)], 0)
                            - _bcast(tt[pl.ds(ei, 16)], 0)) for ei in es]
            tes = _cos_polyN([r_ * wt + bt for r_ in rels])
            accs = [a + qe[ei, pl.ds(128, 16)] * m
                    for a, ei, m in zip(accs, es, msgs)]
            accs = [a + qe[ei, pl.ds(144, 16)] * t_
                    for a, ei, t_ in zip(accs, es, tes)]
            accs = [a + qe[ei, pl.ds(160, 16)]
                    for a, ei in zip(accs, es)]
            exs = [jnp.exp(s_ * INV_SQRT_MEM) for s_ in _allsumN(accs)]
            for ei, exx, m, t_ in zip(es, exs, msgs, tes):
                ov[ei, pl.ds(0, 16)] = exx * m
                ov[ei, pl.ds(16, 16)] = exx * t_
                ov[ei, pl.ds(32, 16)] = exx * unit
            j = e & 15
            for jj in range(4):
                exg = jnp.where(lane == j + jj, exs[jj], exg)

            @pl.when(j == 12)
            def _():
                exall[pl.ds(ci * C + e - 12, 16)] = exg

            return exg

        plsc.parallel_loop(0, C, 4, unroll=1, carry=zeros16)(pair)

    lin_issue(0, 0, 0)
    lin_issue(1, 1, 1)
    lin_drain(0, 0)
    gat_issue(0, 0)

    def quad(q4, carry):
        i0 = 4 * q4
        for j in range(4):
            i = i0 + j
            l4 = j
            k2 = j % 2

            @pl.when(i >= 2)
            def _():
                sca_drain((j + 2) % 4, k2)

            la = lin_issue(i + 2, (j + 2) % 4, k2)
            lin_drain((j + 1) % 4, (j + 1) % 2)
            gat_issue((j + 1) % 4, (j + 1) % 2)
            gat_drain(l4, k2)
            compute(i, l4, k2)
            sca_issue(l4, k2)
            del la
        return carry

    lax.fori_loop(0, NCH // 4, quad, 0)
    sca_drain(2, 0)
    sca_drain(3, 1)
    lin_drain(1, 1)
    gat_drain(0, 0)

    pltpu.sync_copy(exall, ex_hbm.at[wid])

    plsc.subcore_barrier()
    for j in range(NCP):
        r0 = s * RPT + j * CPR
        pltpu.sync_copy(accum.at[pl.ds(r0, CPR)], zb)
        pltpu.sync_copy(zb, out_hbm.at[c, pl.ds(r0, CPR)])


def _run_alpha(ktab, qe, src, dst, t, msg, wt, bt):
    f32 = jnp.float32
    mesh = plsc.VectorSubcoreMesh(core_axis_name="c", subcore_axis_name="s",
                                  num_cores=NC, num_subcores=NS)
    fn = pl.kernel(
        _alpha_body,
        out_type=[jax.ShapeDtypeStruct((NW, EPW), f32),
                  jax.ShapeDtypeStruct((NC, N2, SM_D), f32)],
        mesh=mesh,
        compiler_params=pltpu.CompilerParams(use_tc_tiling_on_sc=False),
        scratch_types=(
            4 * [
                pltpu.VMEM((C,), jnp.int32),
                pltpu.VMEM((C,), jnp.int32),
                pltpu.VMEM((C + 16,), f32),
                pltpu.VMEM((C, 16), f32),
            ]
            + 2 * [
                pltpu.VMEM((C, K_D), f32),
                pltpu.VMEM((C, QE_D), f32),
            ]
            + 2 * [
                pltpu.VMEM((C, SM_D), f32),
            ]
            + [
                pltpu.VMEM((EPW,), f32),
                pltpu.VMEM((1, 16), f32),
                pltpu.VMEM((1, 16), f32),
                pltpu.VMEM((CPR, SM_D), f32),
                pltpu.VMEM_SHARED((N2, SM_D), f32),
            ]
            + 6 * [pltpu.SemaphoreType.DMA]
        ),
    )
    return fn(ktab, qe, src, dst, t, msg, wt, bt)


def _vagg_body(vlo_hbm, vhi_hbm, ex_hbm, src_hbm, dst_hbm, out_hbm,
               srcv0, dstv0, exv0, srcv1, dstv1, exv1,
               srcv2, dstv2, exv2, srcv3, dstv3, exv3,
               vv0, vv1, outv0, outv1, zb, accum,
               semL0, semL1, semG0, semG1, semS0, semS1):
    c = lax.axis_index("c")
    s = lax.axis_index("s")

    srcv = (srcv0, srcv1, srcv2, srcv3)
    dstv = (dstv0, dstv1, dstv2, dstv3)
    exv = (exv0, exv1, exv2, exv3)
    vv = (vv0, vv1)
    outv = (outv0, outv1)
    semL = (semL0, semL1)
    semG = (semG0, semG1)
    semS = (semS0, semS1)

    zeros16 = jnp.zeros((16,), jnp.float32)

    def zrow(j, carry):
        for kk in range(V_D // 16):
            zb[j, pl.ds(kk * 16, 16)] = zeros16
        return carry

    lax.fori_loop(0, CPR, zrow, 0)
    for j in range(NCP):
        pltpu.sync_copy(zb, accum.at[pl.ds(s * RPT + j * CPR, CPR)])
    plsc.subcore_barrier()

    def lin_issue(ci, l4, l2):
        cc = jnp.minimum(ci, NCH2V - 1)
        row = s * 2 + cc // NCHV
        base = (cc % NCHV) * C2
        return [
            pltpu.async_copy(src_hbm.at[row, pl.ds(base, C2)], srcv[l4], semL[l2]),
            pltpu.async_copy(dst_hbm.at[row, pl.ds(base, C2)], dstv[l4], semL[l2]),
            pltpu.async_copy(ex_hbm.at[row, pl.ds(base, C2)], exv[l4].at[pl.ds(0, C2)], semL[l2]),
        ]

    def lin_drain(l4, l2):
        pltpu.make_async_copy(src_hbm.at[0, pl.ds(0, C2)], srcv[l4], semL[l2]).wait()
        pltpu.make_async_copy(dst_hbm.at[0, pl.ds(0, C2)], dstv[l4], semL[l2]).wait()
        pltpu.make_async_copy(ex_hbm.at[0, pl.ds(0, C2)], exv[l4].at[pl.ds(0, C2)], semL[l2]).wait()

    def gat_issue(l4, k2):
        @pl.when(c == 0)
        def _():
            pltpu.async_copy(vlo_hbm.at[srcv[l4]], vv[k2], semG[k2])

        @pl.when(c == 1)
        def _():
            pltpu.async_copy(vhi_hbm.at[srcv[l4]], vv[k2], semG[k2])

    def gat_drain(l4, k2):
        pltpu.make_async_copy(vlo_hbm.at[srcv[l4]], vv[k2], semG[k2]).wait()

    def sca_issue(l4, k2):
        pltpu.async_copy(outv[k2], accum.at[dstv[l4]], semS[k2], add=True)

    def sca_drain(l4, k2):
        pltpu.make_async_copy(outv[k2], accum.at[dstv[l4]], semS[k2]).wait()

    _dn = lax.GatherDimensionNumbers(offset_dims=(), collapsed_slice_dims=(0,),
                                     start_index_map=(0,))

    def _bcast(v, j):
        idx = jnp.full((16, 1), j, jnp.int32)
        return lax.gather(v, idx, _dn, slice_sizes=(1,),
                          mode=lax.GatherScatterMode.PROMISE_IN_BOUNDS)

    def compute(l4, k2):
        ev = exv[l4]
        va = vv[k2]
        ov = outv[k2]

        def pair(e):
            ex0 = _bcast(ev[pl.ds(e, 16)], 0)
            ex1 = _bcast(ev[pl.ds(e + 1, 16)], 0)
            for r in range(V_D // 16):
                ov[e, pl.ds(16 * r, 16)] = ex0 * va[e, pl.ds(16 * r, 16)]
                ov[e + 1, pl.ds(16 * r, 16)] = ex1 * va[e + 1, pl.ds(16 * r, 16)]

        plsc.parallel_loop(0, C2, 2, unroll=2)(pair)

    lin_issue(0, 0, 0)
    lin_issue(1, 1, 1)
    lin_drain(0, 0)
    gat_issue(0, 0)

    def quad(q4, carry):
        i0 = 4 * q4
        for j in range(4):
            i = i0 + j
            l4 = j
            k2 = j % 2

            @pl.when(i >= 2)
            def _():
                sca_drain((j + 2) % 4, k2)

            la = lin_issue(i + 2, (j + 2) % 4, k2)
            lin_drain((j + 1) % 4, (j + 1) % 2)
            gat_issue((j + 1) % 4, (j + 1) % 2)
            gat_drain(l4, k2)
            compute(l4, k2)
            sca_issue(l4, k2)
            del la
        return carry

    lax.fori_loop(0, NCH2V // 4, quad, 0)
    sca_drain(2, 0)
    sca_drain(3, 1)
    lin_drain(1, 1)
    gat_drain(0, 0)

    plsc.subcore_barrier()
    for j in range(NCP):
        r0 = s * RPT + j * CPR
        pltpu.sync_copy(accum.at[pl.ds(r0, CPR)], zb)
        pltpu.sync_copy(zb, out_hbm.at[c, pl.ds(r0, CPR)])


def _run_vagg(vlo, vhi, ex, src, dst):
    f32 = jnp.float32
    mesh = plsc.VectorSubcoreMesh(core_axis_name="c", subcore_axis_name="s",
                                  num_cores=NC, num_subcores=NS)
    fn = pl.kernel(
        _vagg_body,
        out_type=jax.ShapeDtypeStruct((NC, N2, V_D), f32),
        mesh=mesh,
        compiler_params=pltpu.CompilerParams(use_tc_tiling_on_sc=False),
        scratch_types=(
            4 * [
                pltpu.VMEM((C2,), jnp.int32),
                pltpu.VMEM((C2,), jnp.int32),
                pltpu.VMEM((C2 + 16,), f32),
            ]
            + 2 * [
                pltpu.VMEM((C2, V_D), f32),
            ]
            + 2 * [
                pltpu.VMEM((C2, V_D), f32),
            ]
            + [
                pltpu.VMEM((CPR, V_D), f32),
                pltpu.VMEM_SHARED((N2, V_D), f32),
            ]
            + 6 * [pltpu.SemaphoreType.DMA]
        ),
    )
    return fn(vlo, vhi, ex, src, dst)


def _post_body(sm_ref, vagg_ref, enc_ref, We_ref, be_row_ref, Wa_ref,
               ba_row_ref, out_ref):
    f32 = jnp.float32
    S = sm_ref[0] + sm_ref[1]
    Sm = S[:, 0:16]
    St = S[:, 16:32]
    Sd = S[:, 32:33]
    Sv = jnp.concatenate([vagg_ref[0], vagg_ref[1]], axis=1)
    We = We_ref[...]
    dotT = lambda a, w: lax.dot_general(a, w, (((1,), (1,)), ((), ())),
                                        preferred_element_type=f32)
    num = Sv + dotT(Sm, We[:, 0:16]) + dotT(St, We[:, 16:32]) \
        + Sd * be_row_ref[...]
    conv = num / (Sd + 1e-16)
    enc = enc_ref[...]
    Wa = Wa_ref[...]
    lin = dotT(enc, Wa) - jnp.dot(enc, Wa, preferred_element_type=f32) \
        - GAMMA * enc
    h = jnp.tanh(lin + conv + ba_row_ref[...])
    out_ref[...] = jnp.tanh(enc + EPSILON * h)


def _run_post(sm, vagg, enc, We, be_row, Wa, ba_row):
    f32 = jnp.float32
    return pl.pallas_call(
        _post_body,
        grid=(GRID,),
        in_specs=[pl.BlockSpec((NC, BR, SM_D), lambda i: (0, i, 0)),
                  pl.BlockSpec((NC, BR, V_D), lambda i: (0, i, 0)),
                  pl.BlockSpec((BR, MEM), lambda i: (i, 0)),
                  pl.BlockSpec((MEM, 32), lambda i: (0, 0)),
                  pl.BlockSpec((1, MEM), lambda i: (0, 0)),
                  pl.BlockSpec((MEM, MEM), lambda i: (0, 0)),
                  pl.BlockSpec((1, MEM), lambda i: (0, 0))],
        out_specs=pl.BlockSpec((BR, MEM), lambda i: (i, 0)),
        out_shape=jax.ShapeDtypeStruct((N, MEM), f32),
    )(sm, vagg, enc, We, be_row, Wa, ba_row)


def kernel(x, last_update, edge_index, t, msg, W_time, b_time, W_enc, b_enc,
           Wq, bq, Wk, bk, Wv, bv, We, be, W_asym, b_asym):
    PN = N2 - N
    PE = E2 - E
    x2 = jnp.concatenate([x, jnp.zeros((PN, MEM), jnp.float32)], axis=0)
    lu2 = jnp.concatenate([last_update, jnp.zeros((PN,), jnp.float32)]
                          ).reshape(N2, 1)
    row = lambda b: b.reshape(1, MEM)
    ktab, vlo, vhi, qe, enc = _run_pre(x2, lu2, W_enc, row(b_enc), Wq, row(bq),
                                       Wk, row(bk), Wv, row(bv), We,
                                       be.reshape(MEM, 1))
    src = jnp.concatenate([edge_index[0], jnp.zeros((PE,), jnp.int32)]
                          ).reshape(NW, EPW)
    dst = jnp.concatenate([edge_index[1], jnp.full((PE,), N2 - 1, jnp.int32)]
                          ).reshape(NW, EPW)
    t2 = jnp.concatenate([t, jnp.zeros((PE,), jnp.float32)]).reshape(NW, EPW)
    msg2 = jnp.concatenate([msg, jnp.zeros((PE, 16), jnp.float32)], axis=0)
    ex, sm = _run_alpha(ktab, qe, src, dst, t2, msg2,
                        W_time.reshape(1, 16), b_time.reshape(1, 16))
    vagg = _run_vagg(vlo, vhi, ex, src, dst)
    return _run_post(sm, vagg, enc, We, row(be), W_asym, row(b_asym))
